# baseline (speedup 1.0000x reference)
import jax
import jax.numpy as jnp
from jax.experimental import pallas as pl

_N_USERS = 5000
_N_ITEMS = 5000
_N_NODES = _N_USERS + _N_ITEMS
_D = 128


def _combine_body(ego_ref, c1_ref, c2_ref, out_ref):
    out_ref[...] = (ego_ref[...] + c1_ref[...] + c2_ref[...]) * (1.0 / 3.0)


def kernel(ego_embeddings, denoise_user_ids, denoise_item_ids, denoise_treatments, alpha, beta, W1, b1, W2, b2):
    u_emb = ego_embeddings[:_N_USERS]
    i_emb = ego_embeddings[_N_USERS:]
    u_norm = u_emb / jnp.maximum(jnp.linalg.norm(u_emb, axis=1, keepdims=True), 1e-12)
    i_norm = i_emb / jnp.maximum(jnp.linalg.norm(i_emb, axis=1, keepdims=True), 1e-12)
    sim_scores = jnp.sum(u_norm[denoise_user_ids] * i_norm[denoise_item_ids], axis=1)
    e_scores = jax.nn.sigmoid(alpha * sim_scores + beta)
    ps_loss = -jnp.mean(
        denoise_treatments * jnp.log(jnp.clip(e_scores, 1e-12, 1.0))
        + (1.0 - denoise_treatments) * jnp.log(jnp.clip(1.0 - e_scores, 1e-12, 1.0))
    )
    ipw_weights = denoise_treatments / (e_scores + 1e-08)
    rows = jnp.concatenate([denoise_user_ids, denoise_item_ids + _N_USERS])
    cols = jnp.concatenate([denoise_item_ids + _N_USERS, denoise_user_ids])
    vals = jnp.concatenate([ipw_weights, ipw_weights])
    degree = jax.ops.segment_sum(vals, rows, num_segments=_N_NODES) + 1e-08
    d_inv_sqrt = jnp.power(degree, -0.5)
    d_inv_sqrt = jnp.where(jnp.isinf(d_inv_sqrt), 0.0, d_inv_sqrt)
    cur = ego_embeddings
    embs = [cur]
    for (W, b) in ((W1, b1), (W2, b2)):
        msg = jax.ops.segment_sum(vals[:, None] * cur[cols], rows, num_segments=_N_NODES)
        msg = d_inv_sqrt[:, None] * msg
        cur = jax.nn.relu(msg @ W.T + b)
        embs.append(cur)
    denoised = pl.pallas_call(
        _combine_body,
        out_shape=jax.ShapeDtypeStruct((_N_NODES, _D), jnp.float32),
    )(embs[0], embs[1], embs[2])
    return (denoised, ps_loss)


# speedup vs baseline: 6.8122x; 6.8122x over previous
import functools

import jax
import jax.numpy as jnp
from jax import lax
from jax.experimental import pallas as pl
from jax.experimental.pallas import tpu as pltpu
from jax.experimental.pallas import tpu_sc as plsc

_N_USERS = 5000
_N_ITEMS = 5000
_N_NODES = _N_USERS + _N_ITEMS
_N_INTER = 320000
_D = 128
_S_COLS = 5120
_N_PAD = 10240

_NC = 2
_NS = 16
_NW = _NC * _NS
_E_PER_W = _N_INTER // _NW
_CHUNK = 80
_N_CHUNKS = _E_PER_W // _CHUNK
_ROWS_PER_TILE = _N_PAD // _NS


def _k1_body(u_ref, i_ref, ab_ref, s_ref):
    u = u_ref[...]
    it = i_ref[...]
    u_inv = 1.0 / jnp.maximum(jnp.sqrt(jnp.sum(u * u, axis=1, keepdims=True)), 1e-12)
    i_inv = 1.0 / jnp.maximum(jnp.sqrt(jnp.sum(it * it, axis=1, keepdims=True)), 1e-12)
    un = u * u_inv
    inr = it * i_inv
    s = lax.dot_general(un, inr, (((1,), (1,)), ((), ())),
                        preferred_element_type=jnp.float32)
    s_ref[...] = ab_ref[0, 0] * s + ab_ref[0, 1]


def _k1_sim(u_emb, i_emb, ab):
    blk = 512
    grid = (10, 10)
    return pl.pallas_call(
        _k1_body,
        grid=grid,
        in_specs=[
            pl.BlockSpec((blk, _D), lambda i, j: (i, 0)),
            pl.BlockSpec((blk, _D), lambda i, j: (j, 0)),
            pl.BlockSpec(memory_space=pltpu.SMEM),
        ],
        out_specs=pl.BlockSpec((blk, blk), lambda i, j: (i, j)),
        out_shape=jax.ShapeDtypeStruct((_N_USERS, _S_COLS), jnp.float32),
    )(u_emb, i_emb, ab)


def _k2_body(s_flat, uid_hbm, iid_hbm, s_edge_out, uid_v, iid_v, flat_v, s_v, sem):
    wid = lax.axis_index("c") * _NS + lax.axis_index("s")
    base = wid * _E_PER_W

    def chunk(c, carry):
        off = base + c * _CHUNK
        pltpu.sync_copy(uid_hbm.at[pl.ds(off, _CHUNK)], uid_v)
        pltpu.sync_copy(iid_hbm.at[pl.ds(off, _CHUNK)], iid_v)
        for j in range(_CHUNK // 16):
            sl = pl.ds(j * 16, 16)
            flat_v[sl] = uid_v[sl] * _S_COLS + iid_v[sl]
        pltpu.async_copy(s_flat.at[flat_v], s_v, sem).wait()
        pltpu.sync_copy(s_v, s_edge_out.at[pl.ds(off, _CHUNK)])
        return carry

    lax.fori_loop(0, _N_CHUNKS, chunk, 0)


def _k5_body(s_ref, t_ref, ipw_ref, loss_ref):
    s = s_ref[...]
    t = t_ref[...]
    e = jax.nn.sigmoid(s)
    ll = (t * jnp.log(jnp.clip(e, 1e-12, 1.0))
          + (1.0 - t) * jnp.log(jnp.clip(1.0 - e, 1e-12, 1.0)))
    loss_ref[0, 0] = -jnp.sum(ll) * (1.0 / _N_INTER)
    ipw_ref[...] = t / (e + 1e-08)


def _k5_edge_elem(s_edge, treat):
    return pl.pallas_call(
        _k5_body,
        out_shape=(
            jax.ShapeDtypeStruct((_N_INTER // _D, _D), jnp.float32),
            jax.ShapeDtypeStruct((1, 1), jnp.float32),
        ),
        out_specs=(
            pl.BlockSpec((_N_INTER // _D, _D), lambda: (0, 0)),
            pl.BlockSpec(memory_space=pltpu.SMEM),
        ),
    )(s_edge.reshape(_N_INTER // _D, _D), treat.reshape(_N_INTER // _D, _D))


def _k3_body(with_degree, cur_hbm, uid_hbm, iid_hbm, ipw_hbm, msg_out, deg_out,
             uid_v, irow_v, ipw_v, rows_u, rows_i, zbuf, msg_acc, deg_acc,
             sem_u, sem_i):
    cid = lax.axis_index("c")
    sid = lax.axis_index("s")
    base = (cid * _NS + sid) * _E_PER_W
    rbase = sid * _ROWS_PER_TILE

    def zrow(e, carry):
        for j in range(_D // 16):
            zbuf[e, pl.ds(j * 16, 16)] = jnp.zeros((16,), jnp.float32)
        return carry

    lax.fori_loop(0, 128, zrow, 0)
    for k in range(_ROWS_PER_TILE // 128):
        pltpu.sync_copy(zbuf, msg_acc.at[pl.ds(rbase + k * 128, 128)])
        pltpu.sync_copy(zbuf.at[0], deg_acc.at[pl.ds(rbase + k * 128, 128)])
    plsc.subcore_barrier()

    def chunk(c, carry):
        off = base + c * _CHUNK
        pltpu.sync_copy(uid_hbm.at[pl.ds(off, _CHUNK)], uid_v)
        pltpu.sync_copy(iid_hbm.at[pl.ds(off, _CHUNK)], irow_v)
        for j in range(_CHUNK // 16):
            sl = pl.ds(j * 16, 16)
            irow_v[sl] = irow_v[sl] + _N_USERS
        pltpu.sync_copy(ipw_hbm.at[pl.ds(off, _CHUNK)], ipw_v)
        cp_i = pltpu.async_copy(cur_hbm.at[irow_v], rows_i, sem_i)
        cp_u = pltpu.async_copy(cur_hbm.at[uid_v], rows_u, sem_u)
        cp_i.wait()
        cp_u.wait()

        def scale(g, carry2):
            v = ipw_v[pl.ds(g * 16, 16)]
            for e16 in range(16):
                s = v[e16]
                e = g * 16 + e16
                for j in range(_D // 16):
                    sl = pl.ds(j * 16, 16)
                    rows_i[e, sl] = rows_i[e, sl] * s
                    rows_u[e, sl] = rows_u[e, sl] * s
            return carry2

        lax.fori_loop(0, _CHUNK // 16, scale, 0)
        pltpu.sync_copy(rows_i, msg_acc.at[uid_v], add=True)
        pltpu.sync_copy(rows_u, msg_acc.at[irow_v], add=True)
        if with_degree:
            pltpu.sync_copy(ipw_v, deg_acc.at[uid_v], add=True)
            pltpu.sync_copy(ipw_v, deg_acc.at[irow_v], add=True)
        return carry

    lax.fori_loop(0, _N_CHUNKS, chunk, 0)
    plsc.subcore_barrier()

    for k in range(_ROWS_PER_TILE // 128):
        sl = pl.ds(rbase + k * 128, 128)
        pltpu.sync_copy(msg_acc.at[sl], msg_out.at[cid, sl])
        if with_degree:
            pltpu.sync_copy(deg_acc.at[sl], deg_out.at[cid, sl])


@functools.cache
def _sc_kernels():
    mesh = plsc.VectorSubcoreMesh(core_axis_name="c", subcore_axis_name="s",
                                  num_cores=_NC)

    k2 = functools.partial(
        pl.kernel,
        mesh=mesh,
        out_type=jax.ShapeDtypeStruct((_N_INTER,), jnp.float32),
        scratch_types=[
            pltpu.VMEM((_CHUNK,), jnp.int32),
            pltpu.VMEM((_CHUNK,), jnp.int32),
            pltpu.VMEM((_CHUNK,), jnp.int32),
            pltpu.VMEM((_CHUNK,), jnp.float32),
            pltpu.SemaphoreType.DMA,
        ],
    )(_k2_body)

    def make_k3(with_degree):
        out_type = [
            jax.ShapeDtypeStruct((_NC, _N_PAD, _D), jnp.float32),
            jax.ShapeDtypeStruct((_NC, _N_PAD), jnp.float32),
        ]
        return functools.partial(
            pl.kernel,
            mesh=mesh,
            out_type=out_type,
            scratch_types=[
                pltpu.VMEM((_CHUNK,), jnp.int32),
                pltpu.VMEM((_CHUNK,), jnp.int32),
                pltpu.VMEM((_CHUNK,), jnp.float32),
                pltpu.VMEM((_CHUNK, _D), jnp.float32),
                pltpu.VMEM((_CHUNK, _D), jnp.float32),
                pltpu.VMEM((128, _D), jnp.float32),
                pltpu.VMEM_SHARED((_N_PAD, _D), jnp.float32),
                pltpu.VMEM_SHARED((_N_PAD,), jnp.float32),
                pltpu.SemaphoreType.DMA,
                pltpu.SemaphoreType.DMA,
            ],
        )(functools.partial(_k3_body, with_degree))

    return k2, make_k3(True), make_k3(False)


def _k4a_body(deg_ref, msg_ref, w_ref, b_ref, cur_ref, dinv_ref):
    deg = deg_ref[0, :] + deg_ref[1, :] + 1e-08
    dinv = lax.rsqrt(deg)
    dinv = jnp.where(jnp.isinf(dinv), 0.0, dinv)
    m = (msg_ref[0] + msg_ref[1]) * dinv[:, None]
    cur = lax.dot_general(m, w_ref[...], (((1,), (1,)), ((), ())),
                          preferred_element_type=jnp.float32)
    cur_ref[...] = jnp.maximum(cur + b_ref[...], 0.0)
    dinv_ref[...] = dinv[None, :]


def _k4a_layer1(deg_p, msg_p, W, b):
    blk = 1024
    grid = (_N_PAD // blk,)
    return pl.pallas_call(
        _k4a_body,
        grid=grid,
        in_specs=[
            pl.BlockSpec((_NC, blk), lambda r: (0, r)),
            pl.BlockSpec((_NC, blk, _D), lambda r: (0, r, 0)),
            pl.BlockSpec((_D, _D), lambda r: (0, 0)),
            pl.BlockSpec((1, _D), lambda r: (0, 0)),
        ],
        out_specs=(
            pl.BlockSpec((blk, _D), lambda r: (r, 0)),
            pl.BlockSpec((1, blk), lambda r: (0, r)),
        ),
        out_shape=(
            jax.ShapeDtypeStruct((_N_PAD, _D), jnp.float32),
            jax.ShapeDtypeStruct((1, _N_PAD), jnp.float32),
        ),
    )(deg_p, msg_p, W, b)


def _k4b_body(dinv_ref, msg_ref, w_ref, b_ref, ego_ref, cur1_ref, out_ref):
    m = (msg_ref[0] + msg_ref[1]) * dinv_ref[0, :][:, None]
    cur2 = lax.dot_general(m, w_ref[...], (((1,), (1,)), ((), ())),
                           preferred_element_type=jnp.float32)
    cur2 = jnp.maximum(cur2 + b_ref[...], 0.0)
    out_ref[...] = (ego_ref[...] + cur1_ref[...] + cur2) * (1.0 / 3.0)


def _k4b_layer2(dinv, msg_p, W, b, ego_pad, cur1):
    blk = 1024
    grid = (_N_PAD // blk,)
    return pl.pallas_call(
        _k4b_body,
        grid=grid,
        in_specs=[
            pl.BlockSpec((1, blk), lambda r: (0, r)),
            pl.BlockSpec((_NC, blk, _D), lambda r: (0, r, 0)),
            pl.BlockSpec((_D, _D), lambda r: (0, 0)),
            pl.BlockSpec((1, _D), lambda r: (0, 0)),
            pl.BlockSpec((blk, _D), lambda r: (r, 0)),
            pl.BlockSpec((blk, _D), lambda r: (r, 0)),
        ],
        out_specs=pl.BlockSpec((blk, _D), lambda r: (r, 0)),
        out_shape=jax.ShapeDtypeStruct((_N_PAD, _D), jnp.float32),
    )(dinv, msg_p, W, b, ego_pad, cur1)


def kernel(ego_embeddings, denoise_user_ids, denoise_item_ids, denoise_treatments, alpha, beta, W1, b1, W2, b2):
    uid = denoise_user_ids.astype(jnp.int32)
    iid = denoise_item_ids.astype(jnp.int32)
    ab = jnp.stack([alpha, beta]).reshape(1, 2).astype(jnp.float32)

    k2_gather_s, k3_msg_deg, k3_msg = _sc_kernels()

    s_mat = _k1_sim(ego_embeddings[:_N_USERS], ego_embeddings[_N_USERS:], ab)
    s_edge = k2_gather_s(s_mat.reshape(-1), uid, iid)
    ipw2, loss = _k5_edge_elem(s_edge, denoise_treatments)
    ipw = ipw2.reshape(-1)

    ego_pad = jnp.concatenate(
        [ego_embeddings, jnp.zeros((_N_PAD - _N_NODES, _D), jnp.float32)], axis=0)
    msg_p, deg_p = k3_msg_deg(ego_pad, uid, iid, ipw)
    cur1, dinv = _k4a_layer1(deg_p, msg_p, W1, b1.reshape(1, _D))
    msg2_p, _ = k3_msg(cur1, uid, iid, ipw)
    den_pad = _k4b_layer2(dinv, msg2_p, W2, b2.reshape(1, _D), ego_pad, cur1)
    return (den_pad[:_N_NODES], loss.reshape(()))


# speedup vs baseline: 9.0857x; 1.3337x over previous
import functools

import jax
import jax.numpy as jnp
from jax import lax
from jax.experimental import pallas as pl
from jax.experimental.pallas import tpu as pltpu
from jax.experimental.pallas import tpu_sc as plsc

_N_USERS = 5000
_N_ITEMS = 5000
_N_NODES = _N_USERS + _N_ITEMS
_N_INTER = 320000
_D = 128
_S_COLS = 5120
_N_PAD = 10240

_NC = 2
_NS = 16
_NW = _NC * _NS
_HALF = _N_PAD // 2
_CHUNK = 80
_E_PER_W = _N_INTER // _NW
_K2_CHUNKS = _E_PER_W // _CHUNK
_E_PER_TILE = _N_INTER // _NS
_N_CHUNKS = _E_PER_TILE // _CHUNK
_ROWS_PER_TILE = _HALF // _NS


def _k1_body(u_ref, i_ref, ab_ref, s_ref):
    u = u_ref[...]
    it = i_ref[...]
    u_inv = 1.0 / jnp.maximum(jnp.sqrt(jnp.sum(u * u, axis=1, keepdims=True)), 1e-12)
    i_inv = 1.0 / jnp.maximum(jnp.sqrt(jnp.sum(it * it, axis=1, keepdims=True)), 1e-12)
    un = u * u_inv
    inr = it * i_inv
    s = lax.dot_general(un, inr, (((1,), (1,)), ((), ())),
                        preferred_element_type=jnp.float32)
    s_ref[...] = ab_ref[0, 0] * s + ab_ref[0, 1]


def _k1_sim(u_emb, i_emb, ab):
    blk = 512
    grid = (10, 10)
    return pl.pallas_call(
        _k1_body,
        grid=grid,
        in_specs=[
            pl.BlockSpec((blk, _D), lambda i, j: (i, 0)),
            pl.BlockSpec((blk, _D), lambda i, j: (j, 0)),
            pl.BlockSpec(memory_space=pltpu.SMEM),
        ],
        out_specs=pl.BlockSpec((blk, blk), lambda i, j: (i, j)),
        out_shape=jax.ShapeDtypeStruct((_N_USERS, _S_COLS), jnp.float32),
    )(u_emb, i_emb, ab)


def _k2_body(s_flat, uid_hbm, iid_hbm, s_edge_out, uid_v, iid_v, flat_v, s_v, sem):
    wid = lax.axis_index("c") * _NS + lax.axis_index("s")
    base = wid * _E_PER_W

    def chunk(c, carry):
        off = base + c * _CHUNK
        pltpu.sync_copy(uid_hbm.at[pl.ds(off, _CHUNK)], uid_v)
        pltpu.sync_copy(iid_hbm.at[pl.ds(off, _CHUNK)], iid_v)
        for j in range(_CHUNK // 16):
            sl = pl.ds(j * 16, 16)
            flat_v[sl] = uid_v[sl] * _S_COLS + iid_v[sl]
        pltpu.async_copy(s_flat.at[flat_v], s_v, sem).wait()
        pltpu.sync_copy(s_v, s_edge_out.at[pl.ds(off, _CHUNK)])
        return carry

    lax.fori_loop(0, _K2_CHUNKS, chunk, 0)


def _k5_body(s_ref, t_ref, ipw_ref, loss_ref):
    s = s_ref[...]
    t = t_ref[...]
    e = jax.nn.sigmoid(s)
    ll = (t * jnp.log(jnp.clip(e, 1e-12, 1.0))
          + (1.0 - t) * jnp.log(jnp.clip(1.0 - e, 1e-12, 1.0)))
    loss_ref[0, 0] = -jnp.sum(ll) * (1.0 / _N_INTER)
    ipw_ref[...] = t / (e + 1e-08)


def _k5_edge_elem(s_edge, treat):
    shp = (_N_INTER // _D, _D)
    return pl.pallas_call(
        _k5_body,
        out_shape=(
            jax.ShapeDtypeStruct(shp, jnp.float32),
            jax.ShapeDtypeStruct((1, 1), jnp.float32),
        ),
        out_specs=(
            pl.BlockSpec(shp, lambda: (0, 0)),
            pl.BlockSpec(memory_space=pltpu.SMEM),
        ),
    )(s_edge.reshape(shp), treat.reshape(shp))


def _k3_body(with_degree, cur_hbm, uid_hbm, iid_hbm, ipw_hbm, msg_out, deg_out,
             a0, a1, b0, b1, g0, g1, s0, s1, w0, w1, rows0, rows1,
             msg_acc, deg_acc, sem_r0, sem_r1, sem_x0, sem_x1):
    cid = lax.axis_index("c")
    sid = lax.axis_index("s")
    rbase = sid * _ROWS_PER_TILE
    a_v = (a0, a1)
    b_v = (b0, b1)
    g_v = (g0, g1)
    s_v = (s0, s1)
    w_v = (w0, w1)
    rows = (rows0, rows1)
    sem_r = (sem_r0, sem_r1)
    sem_x = (sem_x0, sem_x1)

    def zrow(e, carry):
        for j in range(_D // 16):
            rows0[e, pl.ds(j * 16, 16)] = jnp.zeros((16,), jnp.float32)
        return carry

    lax.fori_loop(0, _CHUNK, zrow, 0)
    for k in range(_ROWS_PER_TILE // _CHUNK):
        pltpu.sync_copy(rows0, msg_acc.at[pl.ds(rbase + k * _CHUNK, _CHUNK)])
    for k in range(_HALF // 128):
        @pl.when(sid == k % _NS)
        def _():
            pltpu.sync_copy(rows0.at[0], deg_acc.at[pl.ds(k * 128, 128)])
    plsc.subcore_barrier()

    def idx_issue(c, b):
        pltpu.async_copy(uid_hbm.at[sid, c], a_v[b], sem_x[b])
        pltpu.async_copy(iid_hbm.at[sid, c], b_v[b], sem_x[b])
        pltpu.async_copy(ipw_hbm.at[sid, c], w_v[b], sem_x[b])

    def idx_wait(c, b):
        pltpu.make_async_copy(uid_hbm.at[sid, c], a_v[b], sem_x[b]).wait()
        pltpu.make_async_copy(iid_hbm.at[sid, c], b_v[b], sem_x[b]).wait()
        pltpu.make_async_copy(ipw_hbm.at[sid, c], w_v[b], sem_x[b]).wait()

    def transform(b):
        @pl.when(cid == 0)
        def _():
            for j in range(_CHUNK // 16):
                sl = pl.ds(j * 16, 16)
                g_v[b][sl] = b_v[b][sl] + _HALF
                s_v[b][sl] = a_v[b][sl]

        @pl.when(cid == 1)
        def _():
            for j in range(_CHUNK // 16):
                sl = pl.ds(j * 16, 16)
                g_v[b][sl] = a_v[b][sl]
                s_v[b][sl] = b_v[b][sl]

    def gather_issue(b):
        pltpu.async_copy(cur_hbm.at[g_v[b]], rows[b], sem_r[b])

    def gather_wait(b):
        pltpu.make_async_copy(cur_hbm.at[g_v[b]], rows[b], sem_r[b]).wait()

    def scale_rows(b):
        def scale(g, carry2):
            v = w_v[b][pl.ds(g * 16, 16)]
            for e16 in range(16):
                s = v[e16]
                e = g * 16 + e16
                for j in range(_D // 16):
                    sl = pl.ds(j * 16, 16)
                    rows[b][e, sl] = rows[b][e, sl] * s
            return carry2

        lax.fori_loop(0, _CHUNK // 16, scale, 0)

    def commit(b):
        pltpu.sync_copy(rows[b], msg_acc.at[s_v[b]], add=True)
        if with_degree:
            pltpu.sync_copy(w_v[b], deg_acc.at[s_v[b]], add=True)

    pltpu.sync_copy(uid_hbm.at[sid, 0], a0)
    pltpu.sync_copy(iid_hbm.at[sid, 0], b0)
    pltpu.sync_copy(ipw_hbm.at[sid, 0], w0)
    transform(0)
    gather_issue(0)
    idx_issue(1, 1)

    def pair(cc, carry):
        for b in range(2):
            c = cc * 2 + b
            q = 1 - b
            gather_wait(b)

            @pl.when(c + 1 < _N_CHUNKS)
            def _():
                idx_wait(c + 1, q)
                transform(q)
                gather_issue(q)

            scale_rows(b)
            commit(b)

            @pl.when(c + 2 < _N_CHUNKS)
            def _():
                idx_issue(c + 2, b)
        return carry

    lax.fori_loop(0, _N_CHUNKS // 2, pair, 0)

    plsc.subcore_barrier()

    obase = cid * _HALF + rbase
    for k in range(_ROWS_PER_TILE // _CHUNK):
        pltpu.sync_copy(msg_acc.at[pl.ds(rbase + k * _CHUNK, _CHUNK)],
                        msg_out.at[pl.ds(obase + k * _CHUNK, _CHUNK)])
    if with_degree:
        for k in range(_HALF // 128):
            @pl.when(sid == k % _NS)
            def _():
                pltpu.sync_copy(deg_acc.at[pl.ds(k * 128, 128)],
                                deg_out.at[pl.ds(cid * _HALF + k * 128, 128)])


@functools.cache
def _sc_kernels():
    mesh = plsc.VectorSubcoreMesh(core_axis_name="c", subcore_axis_name="s",
                                  num_cores=_NC)

    k2 = functools.partial(
        pl.kernel,
        mesh=mesh,
        out_type=jax.ShapeDtypeStruct((_N_INTER,), jnp.float32),
        scratch_types=[
            pltpu.VMEM((_CHUNK,), jnp.int32),
            pltpu.VMEM((_CHUNK,), jnp.int32),
            pltpu.VMEM((_CHUNK,), jnp.int32),
            pltpu.VMEM((_CHUNK,), jnp.float32),
            pltpu.SemaphoreType.DMA,
        ],
    )(_k2_body)

    def make_k3(with_degree):
        out_type = [
            jax.ShapeDtypeStruct((_N_PAD, _D), jnp.float32),
            jax.ShapeDtypeStruct((_N_PAD,), jnp.float32),
        ]
        return functools.partial(
            pl.kernel,
            mesh=mesh,
            out_type=out_type,
            scratch_types=(
                [pltpu.VMEM((_CHUNK,), jnp.int32) for _ in range(8)]
                + [pltpu.VMEM((_CHUNK,), jnp.float32) for _ in range(2)]
                + [
                    pltpu.VMEM((_CHUNK, _D), jnp.float32),
                    pltpu.VMEM((_CHUNK, _D), jnp.float32),
                    pltpu.VMEM_SHARED((_HALF, _D), jnp.float32),
                    pltpu.VMEM_SHARED((_HALF,), jnp.float32),
                    pltpu.SemaphoreType.DMA,
                    pltpu.SemaphoreType.DMA,
                    pltpu.SemaphoreType.DMA,
                    pltpu.SemaphoreType.DMA,
                ]
            ),
        )(functools.partial(_k3_body, with_degree))

    return k2, make_k3(True), make_k3(False)


def _k4a_body(deg_ref, msg_ref, w_ref, b_ref, cur_ref, dinv_ref):
    deg = deg_ref[0, :] + 1e-08
    dinv = lax.rsqrt(deg)
    dinv = jnp.where(jnp.isinf(dinv), 0.0, dinv)
    m = msg_ref[...] * dinv[:, None]
    cur = lax.dot_general(m, w_ref[...], (((1,), (1,)), ((), ())),
                          preferred_element_type=jnp.float32)
    cur_ref[...] = jnp.maximum(cur + b_ref[...], 0.0)
    dinv_ref[...] = dinv[None, :]


def _k4a_layer1(deg, msg, W, b):
    blk = 1024
    grid = (_N_PAD // blk,)
    return pl.pallas_call(
        _k4a_body,
        grid=grid,
        in_specs=[
            pl.BlockSpec((1, blk), lambda r: (0, r)),
            pl.BlockSpec((blk, _D), lambda r: (r, 0)),
            pl.BlockSpec((_D, _D), lambda r: (0, 0)),
            pl.BlockSpec((1, _D), lambda r: (0, 0)),
        ],
        out_specs=(
            pl.BlockSpec((blk, _D), lambda r: (r, 0)),
            pl.BlockSpec((1, blk), lambda r: (0, r)),
        ),
        out_shape=(
            jax.ShapeDtypeStruct((_N_PAD, _D), jnp.float32),
            jax.ShapeDtypeStruct((1, _N_PAD), jnp.float32),
        ),
    )(deg.reshape(1, _N_PAD), msg, W, b)


def _k4b_body(dinv_ref, msg_ref, w_ref, b_ref, ego_ref, cur1_ref, out_ref):
    m = msg_ref[...] * dinv_ref[0, :][:, None]
    cur2 = lax.dot_general(m, w_ref[...], (((1,), (1,)), ((), ())),
                           preferred_element_type=jnp.float32)
    cur2 = jnp.maximum(cur2 + b_ref[...], 0.0)
    out_ref[...] = (ego_ref[...] + cur1_ref[...] + cur2) * (1.0 / 3.0)


def _k4b_layer2(dinv, msg, W, b, ego_pad, cur1):
    blk = 1024
    grid = (_N_PAD // blk,)
    return pl.pallas_call(
        _k4b_body,
        grid=grid,
        in_specs=[
            pl.BlockSpec((1, blk), lambda r: (0, r)),
            pl.BlockSpec((blk, _D), lambda r: (r, 0)),
            pl.BlockSpec((_D, _D), lambda r: (0, 0)),
            pl.BlockSpec((1, _D), lambda r: (0, 0)),
            pl.BlockSpec((blk, _D), lambda r: (r, 0)),
            pl.BlockSpec((blk, _D), lambda r: (r, 0)),
        ],
        out_specs=pl.BlockSpec((blk, _D), lambda r: (r, 0)),
        out_shape=jax.ShapeDtypeStruct((_N_PAD, _D), jnp.float32),
    )(dinv, msg, W, b, ego_pad, cur1)


def kernel(ego_embeddings, denoise_user_ids, denoise_item_ids, denoise_treatments, alpha, beta, W1, b1, W2, b2):
    uid = denoise_user_ids.astype(jnp.int32)
    iid = denoise_item_ids.astype(jnp.int32)
    ab = jnp.stack([alpha, beta]).reshape(1, 2).astype(jnp.float32)

    k2_gather_s, k3_msg_deg, k3_msg = _sc_kernels()

    s_mat = _k1_sim(ego_embeddings[:_N_USERS], ego_embeddings[_N_USERS:], ab)
    s_edge = k2_gather_s(s_mat.reshape(-1), uid, iid)
    ipw2, loss = _k5_edge_elem(s_edge, denoise_treatments)

    eshape = (_NS, _N_CHUNKS, _CHUNK)
    uid3 = uid.reshape(eshape)
    iid3 = iid.reshape(eshape)
    ipw3 = ipw2.reshape(eshape)

    zpad = jnp.zeros((_HALF - _N_USERS, _D), jnp.float32)
    ego_pad = jnp.concatenate(
        [ego_embeddings[:_N_USERS], zpad, ego_embeddings[_N_USERS:], zpad], axis=0)
    msg, deg = k3_msg_deg(ego_pad, uid3, iid3, ipw3)
    cur1, dinv = _k4a_layer1(deg, msg, W1, b1.reshape(1, _D))
    msg2, _ = k3_msg(cur1, uid3, iid3, ipw3)
    den_pad = _k4b_layer2(dinv, msg2, W2, b2.reshape(1, _D), ego_pad, cur1)
    den = jnp.concatenate(
        [den_pad[:_N_USERS], den_pad[_HALF:_HALF + _N_ITEMS]], axis=0)
    return (den, loss.reshape(()))


# speedup vs baseline: 10.0361x; 1.1046x over previous
import functools

import jax
import jax.numpy as jnp
from jax import lax
from jax.experimental import pallas as pl
from jax.experimental.pallas import tpu as pltpu
from jax.experimental.pallas import tpu_sc as plsc

_N_USERS = 5000
_N_ITEMS = 5000
_N_NODES = _N_USERS + _N_ITEMS
_N_INTER = 320000
_D = 128
_S_COLS = 5120
_N_PAD = 10240

_NC = 2
_NS = 16
_NW = _NC * _NS
_HALF = _N_PAD // 2
_CHUNK = 80
_E_PER_W = _N_INTER // _NW
_K2_CHUNKS = _E_PER_W // _CHUNK
_E_PER_TILE = _N_INTER // _NS
_N_CHUNKS = _E_PER_TILE // _CHUNK
_ROWS_PER_TILE = _HALF // _NS


def _k1_body(u_ref, i_ref, ab_ref, s_ref):
    u = u_ref[...]
    it = i_ref[...]
    u_inv = 1.0 / jnp.maximum(jnp.sqrt(jnp.sum(u * u, axis=1, keepdims=True)), 1e-12)
    i_inv = 1.0 / jnp.maximum(jnp.sqrt(jnp.sum(it * it, axis=1, keepdims=True)), 1e-12)
    un = u * u_inv
    inr = it * i_inv
    s = lax.dot_general(un, inr, (((1,), (1,)), ((), ())),
                        preferred_element_type=jnp.float32)
    s_ref[...] = ab_ref[0, 0] * s + ab_ref[0, 1]


def _k1_sim(u_emb, i_emb, ab):
    blk = 512
    grid = (10, 10)
    return pl.pallas_call(
        _k1_body,
        grid=grid,
        in_specs=[
            pl.BlockSpec((blk, _D), lambda i, j: (i, 0)),
            pl.BlockSpec((blk, _D), lambda i, j: (j, 0)),
            pl.BlockSpec(memory_space=pltpu.SMEM),
        ],
        out_specs=pl.BlockSpec((blk, blk), lambda i, j: (i, j)),
        out_shape=jax.ShapeDtypeStruct((_N_USERS, _S_COLS), jnp.float32),
    )(u_emb, i_emb, ab)


def _k2_body(s_flat, uid_hbm, iid_hbm, s_edge_out,
             u0, u1, i0, i1, f0, f1, sv0, sv1, sem_g0, sem_g1, sem_x0, sem_x1):
    wid = lax.axis_index("c") * _NS + lax.axis_index("s")
    base = wid * _E_PER_W
    u_v = (u0, u1)
    i_v = (i0, i1)
    f_v = (f0, f1)
    s_v = (sv0, sv1)
    sem_g = (sem_g0, sem_g1)
    sem_x = (sem_x0, sem_x1)

    def off(c):
        return base + c * _CHUNK

    def idx_issue(c, b):
        pltpu.async_copy(uid_hbm.at[pl.ds(off(c), _CHUNK)], u_v[b], sem_x[b])
        pltpu.async_copy(iid_hbm.at[pl.ds(off(c), _CHUNK)], i_v[b], sem_x[b])

    def idx_wait(c, b):
        pltpu.make_async_copy(uid_hbm.at[pl.ds(off(c), _CHUNK)], u_v[b], sem_x[b]).wait()
        pltpu.make_async_copy(iid_hbm.at[pl.ds(off(c), _CHUNK)], i_v[b], sem_x[b]).wait()

    def flat(b):
        for j in range(_CHUNK // 16):
            sl = pl.ds(j * 16, 16)
            f_v[b][sl] = u_v[b][sl] * _S_COLS + i_v[b][sl]

    def gather_issue(b):
        pltpu.async_copy(s_flat.at[f_v[b]], s_v[b], sem_g[b])

    def gather_wait(b):
        pltpu.make_async_copy(s_flat.at[f_v[b]], s_v[b], sem_g[b]).wait()

    pltpu.sync_copy(uid_hbm.at[pl.ds(off(0), _CHUNK)], u0)
    pltpu.sync_copy(iid_hbm.at[pl.ds(off(0), _CHUNK)], i0)
    flat(0)
    gather_issue(0)
    idx_issue(1, 1)

    def pair(cc, carry):
        for b in range(2):
            c = cc * 2 + b
            q = 1 - b
            gather_wait(b)

            @pl.when(c + 1 < _K2_CHUNKS)
            def _():
                idx_wait(c + 1, q)
                flat(q)
                gather_issue(q)

            pltpu.sync_copy(s_v[b], s_edge_out.at[pl.ds(off(c), _CHUNK)])

            @pl.when(c + 2 < _K2_CHUNKS)
            def _():
                idx_issue(c + 2, b)
        return carry

    lax.fori_loop(0, _K2_CHUNKS // 2, pair, 0)

    c_last = _K2_CHUNKS - 1
    gather_wait(0)
    pltpu.sync_copy(s_v[0], s_edge_out.at[pl.ds(off(c_last), _CHUNK)])


def _k5_body(s_ref, t_ref, ipw_ref, loss_ref):
    s = s_ref[...]
    t = t_ref[...]
    e = jax.nn.sigmoid(s)
    ll = (t * jnp.log(jnp.clip(e, 1e-12, 1.0))
          + (1.0 - t) * jnp.log(jnp.clip(1.0 - e, 1e-12, 1.0)))
    loss_ref[0, 0] = -jnp.sum(ll) * (1.0 / _N_INTER)
    ipw_ref[...] = t / (e + 1e-08)


def _k5_edge_elem(s_edge, treat):
    shp = (_N_INTER // _D, _D)
    return pl.pallas_call(
        _k5_body,
        out_shape=(
            jax.ShapeDtypeStruct(shp, jnp.float32),
            jax.ShapeDtypeStruct((1, 1), jnp.float32),
        ),
        out_specs=(
            pl.BlockSpec(shp, lambda: (0, 0)),
            pl.BlockSpec(memory_space=pltpu.SMEM),
        ),
    )(s_edge.reshape(shp), treat.reshape(shp))


def _k3_body(with_degree, cur_hbm, uid_hbm, iid_hbm, ipw_hbm, msg_out, deg_out,
             a0, a1, b0, b1, g0, g1, s0, s1, w0, w1, rows0, rows1,
             msg_acc, deg_acc, sem_r0, sem_r1, sem_x0, sem_x1):
    cid = lax.axis_index("c")
    sid = lax.axis_index("s")
    rbase = sid * _ROWS_PER_TILE
    a_v = (a0, a1)
    b_v = (b0, b1)
    g_v = (g0, g1)
    s_v = (s0, s1)
    w_v = (w0, w1)
    rows = (rows0, rows1)
    sem_r = (sem_r0, sem_r1)
    sem_x = (sem_x0, sem_x1)

    def zrow(e, carry):
        for j in range(_D // 16):
            rows0[e, pl.ds(j * 16, 16)] = jnp.zeros((16,), jnp.float32)
        return carry

    lax.fori_loop(0, _CHUNK, zrow, 0)
    for k in range(_ROWS_PER_TILE // _CHUNK):
        pltpu.sync_copy(rows0, msg_acc.at[pl.ds(rbase + k * _CHUNK, _CHUNK)])
    for k in range(_HALF // 128):
        @pl.when(sid == k % _NS)
        def _():
            pltpu.sync_copy(rows0.at[0], deg_acc.at[pl.ds(k * 128, 128)])
    plsc.subcore_barrier()

    def idx_issue(c, b):
        pltpu.async_copy(uid_hbm.at[sid, c], a_v[b], sem_x[b])
        pltpu.async_copy(iid_hbm.at[sid, c], b_v[b], sem_x[b])
        pltpu.async_copy(ipw_hbm.at[sid, c], w_v[b], sem_x[b])

    def idx_wait(c, b):
        pltpu.make_async_copy(uid_hbm.at[sid, c], a_v[b], sem_x[b]).wait()
        pltpu.make_async_copy(iid_hbm.at[sid, c], b_v[b], sem_x[b]).wait()
        pltpu.make_async_copy(ipw_hbm.at[sid, c], w_v[b], sem_x[b]).wait()

    def transform(b):
        @pl.when(cid == 0)
        def _():
            for j in range(_CHUNK // 16):
                sl = pl.ds(j * 16, 16)
                g_v[b][sl] = b_v[b][sl] + _HALF
                s_v[b][sl] = a_v[b][sl]

        @pl.when(cid == 1)
        def _():
            for j in range(_CHUNK // 16):
                sl = pl.ds(j * 16, 16)
                g_v[b][sl] = a_v[b][sl]
                s_v[b][sl] = b_v[b][sl]

    def gather_issue(b):
        pltpu.async_copy(cur_hbm.at[g_v[b]], rows[b], sem_r[b])

    def gather_wait(b):
        pltpu.make_async_copy(cur_hbm.at[g_v[b]], rows[b], sem_r[b]).wait()

    def scale_rows(b):
        def scale(g, carry2):
            v = w_v[b][pl.ds(g * 16, 16)]
            for e16 in range(16):
                s = v[e16]
                e = g * 16 + e16
                for j in range(_D // 16):
                    sl = pl.ds(j * 16, 16)
                    rows[b][e, sl] = rows[b][e, sl] * s
            return carry2

        lax.fori_loop(0, _CHUNK // 16, scale, 0)

    def commit(b):
        pltpu.sync_copy(rows[b], msg_acc.at[s_v[b]], add=True)
        if with_degree:
            pltpu.sync_copy(w_v[b], deg_acc.at[s_v[b]], add=True)

    pltpu.sync_copy(uid_hbm.at[sid, 0], a0)
    pltpu.sync_copy(iid_hbm.at[sid, 0], b0)
    pltpu.sync_copy(ipw_hbm.at[sid, 0], w0)
    transform(0)
    gather_issue(0)
    idx_issue(1, 1)

    def pair(cc, carry):
        for b in range(2):
            c = cc * 2 + b
            q = 1 - b
            gather_wait(b)

            @pl.when(c + 1 < _N_CHUNKS)
            def _():
                idx_wait(c + 1, q)
                transform(q)
                gather_issue(q)

            scale_rows(b)
            commit(b)

            @pl.when(c + 2 < _N_CHUNKS)
            def _():
                idx_issue(c + 2, b)
        return carry

    lax.fori_loop(0, _N_CHUNKS // 2, pair, 0)

    plsc.subcore_barrier()

    obase = cid * _HALF + rbase
    for k in range(_ROWS_PER_TILE // _CHUNK):
        pltpu.sync_copy(msg_acc.at[pl.ds(rbase + k * _CHUNK, _CHUNK)],
                        msg_out.at[pl.ds(obase + k * _CHUNK, _CHUNK)])
    if with_degree:
        for k in range(_HALF // 128):
            @pl.when(sid == k % _NS)
            def _():
                pltpu.sync_copy(deg_acc.at[pl.ds(k * 128, 128)],
                                deg_out.at[pl.ds(cid * _HALF + k * 128, 128)])


@functools.cache
def _sc_kernels():
    mesh = plsc.VectorSubcoreMesh(core_axis_name="c", subcore_axis_name="s",
                                  num_cores=_NC)

    k2 = functools.partial(
        pl.kernel,
        mesh=mesh,
        out_type=jax.ShapeDtypeStruct((_N_INTER,), jnp.float32),
        scratch_types=[
            pltpu.VMEM((_CHUNK,), jnp.int32),
            pltpu.VMEM((_CHUNK,), jnp.int32),
            pltpu.VMEM((_CHUNK,), jnp.int32),
            pltpu.VMEM((_CHUNK,), jnp.int32),
            pltpu.VMEM((_CHUNK,), jnp.int32),
            pltpu.VMEM((_CHUNK,), jnp.int32),
            pltpu.VMEM((_CHUNK,), jnp.float32),
            pltpu.VMEM((_CHUNK,), jnp.float32),
            pltpu.SemaphoreType.DMA,
            pltpu.SemaphoreType.DMA,
            pltpu.SemaphoreType.DMA,
            pltpu.SemaphoreType.DMA,
        ],
    )(_k2_body)

    def make_k3(with_degree):
        out_type = [
            jax.ShapeDtypeStruct((_N_PAD, _D), jnp.float32),
            jax.ShapeDtypeStruct((_N_PAD,), jnp.float32),
        ]
        return functools.partial(
            pl.kernel,
            mesh=mesh,
            out_type=out_type,
            scratch_types=(
                [pltpu.VMEM((_CHUNK,), jnp.int32) for _ in range(8)]
                + [pltpu.VMEM((_CHUNK,), jnp.float32) for _ in range(2)]
                + [
                    pltpu.VMEM((_CHUNK, _D), jnp.float32),
                    pltpu.VMEM((_CHUNK, _D), jnp.float32),
                    pltpu.VMEM_SHARED((_HALF, _D), jnp.float32),
                    pltpu.VMEM_SHARED((_HALF,), jnp.float32),
                    pltpu.SemaphoreType.DMA,
                    pltpu.SemaphoreType.DMA,
                    pltpu.SemaphoreType.DMA,
                    pltpu.SemaphoreType.DMA,
                ]
            ),
        )(functools.partial(_k3_body, with_degree))

    return k2, make_k3(True), make_k3(False)


def _k4a_body(deg_ref, msg_ref, w_ref, b_ref, cur_ref, dinv_ref):
    deg = deg_ref[0, :] + 1e-08
    dinv = lax.rsqrt(deg)
    dinv = jnp.where(jnp.isinf(dinv), 0.0, dinv)
    m = msg_ref[...] * dinv[:, None]
    cur = lax.dot_general(m, w_ref[...], (((1,), (1,)), ((), ())),
                          preferred_element_type=jnp.float32)
    cur_ref[...] = jnp.maximum(cur + b_ref[...], 0.0)
    dinv_ref[...] = dinv[None, :]


def _k4a_layer1(deg, msg, W, b):
    blk = 1024
    grid = (_N_PAD // blk,)
    return pl.pallas_call(
        _k4a_body,
        grid=grid,
        in_specs=[
            pl.BlockSpec((1, blk), lambda r: (0, r)),
            pl.BlockSpec((blk, _D), lambda r: (r, 0)),
            pl.BlockSpec((_D, _D), lambda r: (0, 0)),
            pl.BlockSpec((1, _D), lambda r: (0, 0)),
        ],
        out_specs=(
            pl.BlockSpec((blk, _D), lambda r: (r, 0)),
            pl.BlockSpec((1, blk), lambda r: (0, r)),
        ),
        out_shape=(
            jax.ShapeDtypeStruct((_N_PAD, _D), jnp.float32),
            jax.ShapeDtypeStruct((1, _N_PAD), jnp.float32),
        ),
    )(deg.reshape(1, _N_PAD), msg, W, b)


def _k4b_body(dinv_ref, msg_ref, w_ref, b_ref, ego_ref, cur1_ref, out_ref):
    m = msg_ref[...] * dinv_ref[0, :][:, None]
    cur2 = lax.dot_general(m, w_ref[...], (((1,), (1,)), ((), ())),
                           preferred_element_type=jnp.float32)
    cur2 = jnp.maximum(cur2 + b_ref[...], 0.0)
    out_ref[...] = (ego_ref[...] + cur1_ref[...] + cur2) * (1.0 / 3.0)


def _k4b_layer2(dinv, msg, W, b, ego_pad, cur1):
    blk = 1024
    grid = (_N_PAD // blk,)
    return pl.pallas_call(
        _k4b_body,
        grid=grid,
        in_specs=[
            pl.BlockSpec((1, blk), lambda r: (0, r)),
            pl.BlockSpec((blk, _D), lambda r: (r, 0)),
            pl.BlockSpec((_D, _D), lambda r: (0, 0)),
            pl.BlockSpec((1, _D), lambda r: (0, 0)),
            pl.BlockSpec((blk, _D), lambda r: (r, 0)),
            pl.BlockSpec((blk, _D), lambda r: (r, 0)),
        ],
        out_specs=pl.BlockSpec((blk, _D), lambda r: (r, 0)),
        out_shape=jax.ShapeDtypeStruct((_N_PAD, _D), jnp.float32),
    )(dinv, msg, W, b, ego_pad, cur1)


def kernel(ego_embeddings, denoise_user_ids, denoise_item_ids, denoise_treatments, alpha, beta, W1, b1, W2, b2):
    uid = denoise_user_ids.astype(jnp.int32)
    iid = denoise_item_ids.astype(jnp.int32)
    ab = jnp.stack([alpha, beta]).reshape(1, 2).astype(jnp.float32)

    k2_gather_s, k3_msg_deg, k3_msg = _sc_kernels()

    s_mat = _k1_sim(ego_embeddings[:_N_USERS], ego_embeddings[_N_USERS:], ab)
    s_edge = k2_gather_s(s_mat.reshape(-1), uid, iid)
    ipw2, loss = _k5_edge_elem(s_edge, denoise_treatments)

    eshape = (_NS, _N_CHUNKS, _CHUNK)
    uid3 = uid.reshape(eshape)
    iid3 = iid.reshape(eshape)
    ipw3 = ipw2.reshape(eshape)

    zpad = jnp.zeros((_HALF - _N_USERS, _D), jnp.float32)
    ego_pad = jnp.concatenate(
        [ego_embeddings[:_N_USERS], zpad, ego_embeddings[_N_USERS:], zpad], axis=0)
    msg, deg = k3_msg_deg(ego_pad, uid3, iid3, ipw3)
    cur1, dinv = _k4a_layer1(deg, msg, W1, b1.reshape(1, _D))
    msg2, _ = k3_msg(cur1, uid3, iid3, ipw3)
    den_pad = _k4b_layer2(dinv, msg2, W2, b2.reshape(1, _D), ego_pad, cur1)
    den = jnp.concatenate(
        [den_pad[:_N_USERS], den_pad[_HALF:_HALF + _N_ITEMS]], axis=0)
    return (den, loss.reshape(()))


# speedup vs baseline: 11.5348x; 1.1493x over previous
import functools

import jax
import jax.numpy as jnp
from jax import lax
from jax.experimental import pallas as pl
from jax.experimental.pallas import tpu as pltpu
from jax.experimental.pallas import tpu_sc as plsc

_N_USERS = 5000
_N_ITEMS = 5000
_N_NODES = _N_USERS + _N_ITEMS
_N_INTER = 320000
_D = 128
_S_COLS = 5120
_N_PAD = 10240

_NC = 2
_NS = 16
_NW = _NC * _NS
_HALF = _N_PAD // 2
_CHUNK = 80
_E_PER_W = _N_INTER // _NW
_K2_CHUNKS = _E_PER_W // _CHUNK
_E_PER_TILE = _N_INTER // _NS
_N_CHUNKS = _E_PER_TILE // _CHUNK
_ROWS_PER_TILE = _HALF // _NS


def _k1_body(u_ref, i_ref, ab_ref, s_ref):
    u = u_ref[...]
    it = i_ref[...]
    u_inv = 1.0 / jnp.maximum(jnp.sqrt(jnp.sum(u * u, axis=1, keepdims=True)), 1e-12)
    i_inv = 1.0 / jnp.maximum(jnp.sqrt(jnp.sum(it * it, axis=1, keepdims=True)), 1e-12)
    un = u * u_inv
    inr = it * i_inv
    s = lax.dot_general(un, inr, (((1,), (1,)), ((), ())),
                        preferred_element_type=jnp.float32)
    s_ref[...] = ab_ref[0, 0] * s + ab_ref[0, 1]


def _k1_sim(u_emb, i_emb, ab):
    blk = 512
    grid = (10, 10)
    return pl.pallas_call(
        _k1_body,
        grid=grid,
        in_specs=[
            pl.BlockSpec((blk, _D), lambda i, j: (i, 0)),
            pl.BlockSpec((blk, _D), lambda i, j: (j, 0)),
            pl.BlockSpec(memory_space=pltpu.SMEM),
        ],
        out_specs=pl.BlockSpec((blk, blk), lambda i, j: (i, j)),
        out_shape=jax.ShapeDtypeStruct((_N_USERS, _S_COLS), jnp.float32),
    )(u_emb, i_emb, ab)


def _k2_body(s_flat, uid_hbm, iid_hbm, s_edge_out,
             u0, u1, i0, i1, f0, f1, sv0, sv1, sem_g0, sem_g1, sem_x0, sem_x1):
    wid = lax.axis_index("c") * _NS + lax.axis_index("s")
    base = wid * _E_PER_W
    u_v = (u0, u1)
    i_v = (i0, i1)
    f_v = (f0, f1)
    s_v = (sv0, sv1)
    sem_g = (sem_g0, sem_g1)
    sem_x = (sem_x0, sem_x1)

    def off(c):
        return base + c * _CHUNK

    def idx_issue(c, b):
        pltpu.async_copy(uid_hbm.at[pl.ds(off(c), _CHUNK)], u_v[b], sem_x[b])
        pltpu.async_copy(iid_hbm.at[pl.ds(off(c), _CHUNK)], i_v[b], sem_x[b])

    def idx_wait(c, b):
        pltpu.make_async_copy(uid_hbm.at[pl.ds(off(c), _CHUNK)], u_v[b], sem_x[b]).wait()
        pltpu.make_async_copy(iid_hbm.at[pl.ds(off(c), _CHUNK)], i_v[b], sem_x[b]).wait()

    def flat(b):
        for j in range(_CHUNK // 16):
            sl = pl.ds(j * 16, 16)
            f_v[b][sl] = u_v[b][sl] * _S_COLS + i_v[b][sl]

    def gather_issue(b):
        pltpu.async_copy(s_flat.at[f_v[b]], s_v[b], sem_g[b])

    def gather_wait(b):
        pltpu.make_async_copy(s_flat.at[f_v[b]], s_v[b], sem_g[b]).wait()

    pltpu.sync_copy(uid_hbm.at[pl.ds(off(0), _CHUNK)], u0)
    pltpu.sync_copy(iid_hbm.at[pl.ds(off(0), _CHUNK)], i0)
    flat(0)
    gather_issue(0)
    idx_issue(1, 1)

    def pair(cc, carry):
        for b in range(2):
            c = cc * 2 + b
            q = 1 - b
            gather_wait(b)

            @pl.when(c + 1 < _K2_CHUNKS)
            def _():
                idx_wait(c + 1, q)
                flat(q)
                gather_issue(q)

            pltpu.sync_copy(s_v[b], s_edge_out.at[pl.ds(off(c), _CHUNK)])

            @pl.when(c + 2 < _K2_CHUNKS)
            def _():
                idx_issue(c + 2, b)
        return carry

    lax.fori_loop(0, _K2_CHUNKS // 2, pair, 0)

    c_last = _K2_CHUNKS - 1
    gather_wait(0)
    pltpu.sync_copy(s_v[0], s_edge_out.at[pl.ds(off(c_last), _CHUNK)])


def _k5_body(s_ref, t_ref, ipw_ref, loss_ref):
    s = s_ref[...]
    t = t_ref[...]
    e = jax.nn.sigmoid(s)
    ll = (t * jnp.log(jnp.clip(e, 1e-12, 1.0))
          + (1.0 - t) * jnp.log(jnp.clip(1.0 - e, 1e-12, 1.0)))
    loss_ref[0, 0] = -jnp.sum(ll) * (1.0 / _N_INTER)
    ipw_ref[...] = t / (e + 1e-08)


def _k5_edge_elem(s_edge, treat):
    shp = (_N_INTER // _D, _D)
    return pl.pallas_call(
        _k5_body,
        out_shape=(
            jax.ShapeDtypeStruct(shp, jnp.float32),
            jax.ShapeDtypeStruct((1, 1), jnp.float32),
        ),
        out_specs=(
            pl.BlockSpec(shp, lambda: (0, 0)),
            pl.BlockSpec(memory_space=pltpu.SMEM),
        ),
    )(s_edge.reshape(shp), treat.reshape(shp))


def _k3_body(with_degree, cur_hbm, uid_hbm, iid_hbm, ipw_hbm, msg_out, deg_out,
             a0, a1, b0, b1, g0, g1, s0, s1, w0, w1, wd0, wd1, rows0, rows1,
             msg_acc, deg_acc, sem_r0, sem_r1, sem_x0, sem_x1, sem_c0, sem_c1):
    cid = lax.axis_index("c")
    sid = lax.axis_index("s")
    rbase = sid * _ROWS_PER_TILE
    a_v = (a0, a1)
    b_v = (b0, b1)
    g_v = (g0, g1)
    s_v = (s0, s1)
    w_v = (w0, w1)
    rows = (rows0, rows1)
    sem_r = (sem_r0, sem_r1)
    sem_x = (sem_x0, sem_x1)

    def zrow(e, carry):
        for j in range(_D // 16):
            rows0[e, pl.ds(j * 16, 16)] = jnp.zeros((16,), jnp.float32)
        return carry

    lax.fori_loop(0, _CHUNK, zrow, 0)
    for k in range(_ROWS_PER_TILE // _CHUNK):
        pltpu.sync_copy(rows0, msg_acc.at[pl.ds(rbase + k * _CHUNK, _CHUNK)])
    for k in range(_HALF // 128):
        @pl.when(sid == k % _NS)
        def _():
            pltpu.sync_copy(rows0.at[0], deg_acc.at[pl.ds(k * 128, 128)])
    plsc.subcore_barrier()

    def idx_issue(c, b):
        pltpu.async_copy(uid_hbm.at[sid, c], a_v[b], sem_x[b])
        pltpu.async_copy(iid_hbm.at[sid, c], b_v[b], sem_x[b])
        pltpu.async_copy(ipw_hbm.at[sid, c], w_v[b], sem_x[b])

    def idx_wait(c, b):
        pltpu.make_async_copy(uid_hbm.at[sid, c], a_v[b], sem_x[b]).wait()
        pltpu.make_async_copy(iid_hbm.at[sid, c], b_v[b], sem_x[b]).wait()
        pltpu.make_async_copy(ipw_hbm.at[sid, c], w_v[b], sem_x[b]).wait()

    def transform(b):
        @pl.when(cid == 0)
        def _():
            for j in range(_CHUNK // 16):
                sl = pl.ds(j * 16, 16)
                g_v[b][sl] = b_v[b][sl] + _HALF
                s_v[b][sl] = a_v[b][sl]

        @pl.when(cid == 1)
        def _():
            for j in range(_CHUNK // 16):
                sl = pl.ds(j * 16, 16)
                g_v[b][sl] = a_v[b][sl]
                s_v[b][sl] = b_v[b][sl]

    def gather_issue(b):
        pltpu.async_copy(cur_hbm.at[g_v[b]], rows[b], sem_r[b])

    def gather_wait(b):
        pltpu.make_async_copy(cur_hbm.at[g_v[b]], rows[b], sem_r[b]).wait()

    def scale_rows(b):
        def scale(g, carry2):
            v = w_v[b][pl.ds(g * 16, 16)]
            for e16 in range(16):
                s = v[e16]
                e = g * 16 + e16
                for j in range(_D // 16):
                    sl = pl.ds(j * 16, 16)
                    rows[b][e, sl] = rows[b][e, sl] * s
            return carry2

        lax.fori_loop(0, _CHUNK // 16, scale, 0)

    sem_c = (sem_c0, sem_c1)
    wd_v = (wd0, wd1)

    def commit_issue(b):
        pltpu.async_copy(rows[b], msg_acc.at[s_v[b]], sem_c[b], add=True)
        if with_degree:
            for j in range(_CHUNK // 16):
                sl = pl.ds(j * 16, 16)
                wd_v[b][sl] = w_v[b][sl]
            pltpu.async_copy(wd_v[b], deg_acc.at[s_v[b]], sem_c[b], add=True)

    def commit_wait(b):
        pltpu.make_async_copy(rows[b], msg_acc.at[s_v[b]], sem_c[b]).wait()
        if with_degree:
            pltpu.make_async_copy(wd_v[b], deg_acc.at[s_v[b]], sem_c[b]).wait()

    pltpu.sync_copy(uid_hbm.at[sid, 0], a0)
    pltpu.sync_copy(iid_hbm.at[sid, 0], b0)
    pltpu.sync_copy(ipw_hbm.at[sid, 0], w0)
    transform(0)
    gather_issue(0)
    idx_issue(1, 1)

    def pair(cc, carry):
        for b in range(2):
            c = cc * 2 + b
            q = 1 - b
            gather_wait(b)

            @pl.when(c + 1 < _N_CHUNKS)
            def _():
                idx_wait(c + 1, q)

                @pl.when(c >= 1)
                def _():
                    commit_wait(q)

                transform(q)
                gather_issue(q)

            scale_rows(b)
            commit_issue(b)

            @pl.when(c + 2 < _N_CHUNKS)
            def _():
                idx_issue(c + 2, b)
        return carry

    lax.fori_loop(0, _N_CHUNKS // 2, pair, 0)

    commit_wait(0)
    commit_wait(1)
    plsc.subcore_barrier()

    obase = cid * _HALF + rbase
    for k in range(_ROWS_PER_TILE // _CHUNK):
        pltpu.sync_copy(msg_acc.at[pl.ds(rbase + k * _CHUNK, _CHUNK)],
                        msg_out.at[pl.ds(obase + k * _CHUNK, _CHUNK)])
    if with_degree:
        for k in range(_HALF // 128):
            @pl.when(sid == k % _NS)
            def _():
                pltpu.sync_copy(deg_acc.at[pl.ds(k * 128, 128)],
                                deg_out.at[pl.ds(cid * _HALF + k * 128, 128)])


@functools.cache
def _sc_kernels():
    mesh = plsc.VectorSubcoreMesh(core_axis_name="c", subcore_axis_name="s",
                                  num_cores=_NC)

    k2 = functools.partial(
        pl.kernel,
        mesh=mesh,
        out_type=jax.ShapeDtypeStruct((_N_INTER,), jnp.float32),
        scratch_types=[
            pltpu.VMEM((_CHUNK,), jnp.int32),
            pltpu.VMEM((_CHUNK,), jnp.int32),
            pltpu.VMEM((_CHUNK,), jnp.int32),
            pltpu.VMEM((_CHUNK,), jnp.int32),
            pltpu.VMEM((_CHUNK,), jnp.int32),
            pltpu.VMEM((_CHUNK,), jnp.int32),
            pltpu.VMEM((_CHUNK,), jnp.float32),
            pltpu.VMEM((_CHUNK,), jnp.float32),
            pltpu.SemaphoreType.DMA,
            pltpu.SemaphoreType.DMA,
            pltpu.SemaphoreType.DMA,
            pltpu.SemaphoreType.DMA,
        ],
    )(_k2_body)

    def make_k3(with_degree):
        out_type = [
            jax.ShapeDtypeStruct((_N_PAD, _D), jnp.float32),
            jax.ShapeDtypeStruct((_N_PAD,), jnp.float32),
        ]
        return functools.partial(
            pl.kernel,
            mesh=mesh,
            out_type=out_type,
            scratch_types=(
                [pltpu.VMEM((_CHUNK,), jnp.int32) for _ in range(8)]
                + [pltpu.VMEM((_CHUNK,), jnp.float32) for _ in range(4)]
                + [
                    pltpu.VMEM((_CHUNK, _D), jnp.float32),
                    pltpu.VMEM((_CHUNK, _D), jnp.float32),
                    pltpu.VMEM_SHARED((_HALF, _D), jnp.float32),
                    pltpu.VMEM_SHARED((_HALF,), jnp.float32),
                    pltpu.SemaphoreType.DMA,
                    pltpu.SemaphoreType.DMA,
                    pltpu.SemaphoreType.DMA,
                    pltpu.SemaphoreType.DMA,
                    pltpu.SemaphoreType.DMA,
                    pltpu.SemaphoreType.DMA,
                ]
            ),
        )(functools.partial(_k3_body, with_degree))

    return k2, make_k3(True), make_k3(False)


def _k4a_body(deg_ref, msg_ref, w_ref, b_ref, cur_ref, dinv_ref):
    deg = deg_ref[0, :] + 1e-08
    dinv = lax.rsqrt(deg)
    dinv = jnp.where(jnp.isinf(dinv), 0.0, dinv)
    m = msg_ref[...] * dinv[:, None]
    cur = lax.dot_general(m, w_ref[...], (((1,), (1,)), ((), ())),
                          preferred_element_type=jnp.float32)
    cur_ref[...] = jnp.maximum(cur + b_ref[...], 0.0)
    dinv_ref[...] = dinv[None, :]


def _k4a_layer1(deg, msg, W, b):
    blk = 1024
    grid = (_N_PAD // blk,)
    return pl.pallas_call(
        _k4a_body,
        grid=grid,
        in_specs=[
            pl.BlockSpec((1, blk), lambda r: (0, r)),
            pl.BlockSpec((blk, _D), lambda r: (r, 0)),
            pl.BlockSpec((_D, _D), lambda r: (0, 0)),
            pl.BlockSpec((1, _D), lambda r: (0, 0)),
        ],
        out_specs=(
            pl.BlockSpec((blk, _D), lambda r: (r, 0)),
            pl.BlockSpec((1, blk), lambda r: (0, r)),
        ),
        out_shape=(
            jax.ShapeDtypeStruct((_N_PAD, _D), jnp.float32),
            jax.ShapeDtypeStruct((1, _N_PAD), jnp.float32),
        ),
    )(deg.reshape(1, _N_PAD), msg, W, b)


def _k4b_body(dinv_ref, msg_ref, w_ref, b_ref, ego_ref, cur1_ref, out_ref):
    m = msg_ref[...] * dinv_ref[0, :][:, None]
    cur2 = lax.dot_general(m, w_ref[...], (((1,), (1,)), ((), ())),
                           preferred_element_type=jnp.float32)
    cur2 = jnp.maximum(cur2 + b_ref[...], 0.0)
    out_ref[...] = (ego_ref[...] + cur1_ref[...] + cur2) * (1.0 / 3.0)


def _k4b_layer2(dinv, msg, W, b, ego_pad, cur1):
    blk = 1024
    grid = (_N_PAD // blk,)
    return pl.pallas_call(
        _k4b_body,
        grid=grid,
        in_specs=[
            pl.BlockSpec((1, blk), lambda r: (0, r)),
            pl.BlockSpec((blk, _D), lambda r: (r, 0)),
            pl.BlockSpec((_D, _D), lambda r: (0, 0)),
            pl.BlockSpec((1, _D), lambda r: (0, 0)),
            pl.BlockSpec((blk, _D), lambda r: (r, 0)),
            pl.BlockSpec((blk, _D), lambda r: (r, 0)),
        ],
        out_specs=pl.BlockSpec((blk, _D), lambda r: (r, 0)),
        out_shape=jax.ShapeDtypeStruct((_N_PAD, _D), jnp.float32),
    )(dinv, msg, W, b, ego_pad, cur1)


def kernel(ego_embeddings, denoise_user_ids, denoise_item_ids, denoise_treatments, alpha, beta, W1, b1, W2, b2):
    uid = denoise_user_ids.astype(jnp.int32)
    iid = denoise_item_ids.astype(jnp.int32)
    ab = jnp.stack([alpha, beta]).reshape(1, 2).astype(jnp.float32)

    k2_gather_s, k3_msg_deg, k3_msg = _sc_kernels()

    s_mat = _k1_sim(ego_embeddings[:_N_USERS], ego_embeddings[_N_USERS:], ab)
    s_edge = k2_gather_s(s_mat.reshape(-1), uid, iid)
    ipw2, loss = _k5_edge_elem(s_edge, denoise_treatments)

    eshape = (_NS, _N_CHUNKS, _CHUNK)
    uid3 = uid.reshape(eshape)
    iid3 = iid.reshape(eshape)
    ipw3 = ipw2.reshape(eshape)

    zpad = jnp.zeros((_HALF - _N_USERS, _D), jnp.float32)
    ego_pad = jnp.concatenate(
        [ego_embeddings[:_N_USERS], zpad, ego_embeddings[_N_USERS:], zpad], axis=0)
    msg, deg = k3_msg_deg(ego_pad, uid3, iid3, ipw3)
    cur1, dinv = _k4a_layer1(deg, msg, W1, b1.reshape(1, _D))
    msg2, _ = k3_msg(cur1, uid3, iid3, ipw3)
    den_pad = _k4b_layer2(dinv, msg2, W2, b2.reshape(1, _D), ego_pad, cur1)
    den = jnp.concatenate(
        [den_pad[:_N_USERS], den_pad[_HALF:_HALF + _N_ITEMS]], axis=0)
    return (den, loss.reshape(()))


# speedup vs baseline: 11.5359x; 1.0001x over previous
import functools

import jax
import jax.numpy as jnp
from jax import lax
from jax.experimental import pallas as pl
from jax.experimental.pallas import tpu as pltpu
from jax.experimental.pallas import tpu_sc as plsc

_N_USERS = 5000
_N_ITEMS = 5000
_N_NODES = _N_USERS + _N_ITEMS
_N_INTER = 320000
_D = 128
_S_COLS = 5120
_N_PAD = 10240

_NC = 2
_NS = 16
_NW = _NC * _NS
_HALF = _N_PAD // 2
_CHUNK = 80
_E_PER_W = _N_INTER // _NW
_K2_CHUNKS = _E_PER_W // _CHUNK
_E_PER_TILE = _N_INTER // _NS
_N_CHUNKS = _E_PER_TILE // _CHUNK
_ROWS_PER_TILE = _HALF // _NS


def _k1_body(u_ref, i_ref, ab_ref, s_ref):
    u = u_ref[...]
    it = i_ref[...]
    u_inv = 1.0 / jnp.maximum(jnp.sqrt(jnp.sum(u * u, axis=1, keepdims=True)), 1e-12)
    i_inv = 1.0 / jnp.maximum(jnp.sqrt(jnp.sum(it * it, axis=1, keepdims=True)), 1e-12)
    un = u * u_inv
    inr = it * i_inv
    s = lax.dot_general(un, inr, (((1,), (1,)), ((), ())),
                        preferred_element_type=jnp.float32)
    s_ref[...] = ab_ref[0, 0] * s + ab_ref[0, 1]


def _k1_sim(u_emb, i_emb, ab):
    blk = 512
    grid = (10, 10)
    return pl.pallas_call(
        _k1_body,
        grid=grid,
        in_specs=[
            pl.BlockSpec((blk, _D), lambda i, j: (i, 0)),
            pl.BlockSpec((blk, _D), lambda i, j: (j, 0)),
            pl.BlockSpec(memory_space=pltpu.SMEM),
        ],
        out_specs=pl.BlockSpec((blk, blk), lambda i, j: (i, j)),
        out_shape=jax.ShapeDtypeStruct((_N_USERS, _S_COLS), jnp.float32),
    )(u_emb, i_emb, ab)


def _k2_body(s_flat, uid_hbm, iid_hbm, s_edge_out,
             u0, u1, i0, i1, f0, f1, sv0, sv1, sem_g0, sem_g1, sem_x0, sem_x1):
    wid = lax.axis_index("c") * _NS + lax.axis_index("s")
    base = wid * _E_PER_W
    u_v = (u0, u1)
    i_v = (i0, i1)
    f_v = (f0, f1)
    s_v = (sv0, sv1)
    sem_g = (sem_g0, sem_g1)
    sem_x = (sem_x0, sem_x1)

    def off(c):
        return base + c * _CHUNK

    def idx_issue(c, b):
        pltpu.async_copy(uid_hbm.at[pl.ds(off(c), _CHUNK)], u_v[b], sem_x[b])
        pltpu.async_copy(iid_hbm.at[pl.ds(off(c), _CHUNK)], i_v[b], sem_x[b])

    def idx_wait(c, b):
        pltpu.make_async_copy(uid_hbm.at[pl.ds(off(c), _CHUNK)], u_v[b], sem_x[b]).wait()
        pltpu.make_async_copy(iid_hbm.at[pl.ds(off(c), _CHUNK)], i_v[b], sem_x[b]).wait()

    def flat(b):
        for j in range(_CHUNK // 16):
            sl = pl.ds(j * 16, 16)
            f_v[b][sl] = u_v[b][sl] * _S_COLS + i_v[b][sl]

    def gather_issue(b):
        pltpu.async_copy(s_flat.at[f_v[b]], s_v[b], sem_g[b])

    def gather_wait(b):
        pltpu.make_async_copy(s_flat.at[f_v[b]], s_v[b], sem_g[b]).wait()

    pltpu.sync_copy(uid_hbm.at[pl.ds(off(0), _CHUNK)], u0)
    pltpu.sync_copy(iid_hbm.at[pl.ds(off(0), _CHUNK)], i0)
    flat(0)
    gather_issue(0)
    idx_issue(1, 1)

    def pair(cc, carry):
        for b in range(2):
            c = cc * 2 + b
            q = 1 - b
            gather_wait(b)

            @pl.when(c + 1 < _K2_CHUNKS)
            def _():
                idx_wait(c + 1, q)
                flat(q)
                gather_issue(q)

            pltpu.sync_copy(s_v[b], s_edge_out.at[pl.ds(off(c), _CHUNK)])

            @pl.when(c + 2 < _K2_CHUNKS)
            def _():
                idx_issue(c + 2, b)
        return carry

    lax.fori_loop(0, _K2_CHUNKS // 2, pair, 0)

    c_last = _K2_CHUNKS - 1
    gather_wait(0)
    pltpu.sync_copy(s_v[0], s_edge_out.at[pl.ds(off(c_last), _CHUNK)])


def _k5_body(s_ref, t_ref, ipw_ref, loss_ref):
    s = s_ref[...]
    t = t_ref[...]
    e = jax.nn.sigmoid(s)
    ll = (t * jnp.log(jnp.clip(e, 1e-12, 1.0))
          + (1.0 - t) * jnp.log(jnp.clip(1.0 - e, 1e-12, 1.0)))
    loss_ref[0, 0] = -jnp.sum(ll) * (1.0 / _N_INTER)
    ipw_ref[...] = t / (e + 1e-08)


def _k5_edge_elem(s_edge, treat):
    shp = (_N_INTER // _D, _D)
    return pl.pallas_call(
        _k5_body,
        out_shape=(
            jax.ShapeDtypeStruct(shp, jnp.float32),
            jax.ShapeDtypeStruct((1, 1), jnp.float32),
        ),
        out_specs=(
            pl.BlockSpec(shp, lambda: (0, 0)),
            pl.BlockSpec(memory_space=pltpu.SMEM),
        ),
    )(s_edge.reshape(shp), treat.reshape(shp))


def _k3_body(with_degree, cur_hbm, meta_hbm, msg_out, deg_out,
             m0, m1, g0, g1, s0, s1, wd0, wd1, rows0, rows1,
             msg_acc, deg_acc, sem_r0, sem_r1, sem_x0, sem_x1, sem_c0, sem_c1):
    cid = lax.axis_index("c")
    sid = lax.axis_index("s")
    rbase = sid * _ROWS_PER_TILE
    m_v = (m0, m1)
    g_v = (g0, g1)
    s_v = (s0, s1)
    rows = (rows0, rows1)
    sem_r = (sem_r0, sem_r1)
    sem_x = (sem_x0, sem_x1)

    def zrow(e, carry):
        for j in range(_D // 16):
            rows0[e, pl.ds(j * 16, 16)] = jnp.zeros((16,), jnp.float32)
        return carry

    lax.fori_loop(0, _CHUNK, zrow, 0)
    for k in range(_ROWS_PER_TILE // _CHUNK):
        pltpu.sync_copy(rows0, msg_acc.at[pl.ds(rbase + k * _CHUNK, _CHUNK)])
    for k in range(_HALF // 128):
        @pl.when(sid == k % _NS)
        def _():
            pltpu.sync_copy(rows0.at[0], deg_acc.at[pl.ds(k * 128, 128)])
    plsc.subcore_barrier()

    def idx_issue(c, b):
        pltpu.async_copy(meta_hbm.at[sid, c], m_v[b], sem_x[b])

    def idx_wait(c, b):
        pltpu.make_async_copy(meta_hbm.at[sid, c], m_v[b], sem_x[b]).wait()

    def transform(b):
        @pl.when(cid == 0)
        def _():
            for j in range(_CHUNK // 16):
                sl = pl.ds(j * 16, 16)
                g_v[b][sl] = m_v[b][pl.ds(_CHUNK + j * 16, 16)] + _HALF
                s_v[b][sl] = m_v[b][sl]

        @pl.when(cid == 1)
        def _():
            for j in range(_CHUNK // 16):
                sl = pl.ds(j * 16, 16)
                g_v[b][sl] = m_v[b][sl]
                s_v[b][sl] = m_v[b][pl.ds(_CHUNK + j * 16, 16)]

    def gather_issue(b):
        pltpu.async_copy(cur_hbm.at[g_v[b]], rows[b], sem_r[b])

    def gather_wait(b):
        pltpu.make_async_copy(cur_hbm.at[g_v[b]], rows[b], sem_r[b]).wait()

    def scale_rows(b):
        def scale(g, carry2):
            v = lax.bitcast_convert_type(
                m_v[b][pl.ds(2 * _CHUNK + g * 16, 16)], jnp.float32)
            for e16 in range(16):
                s = v[e16]
                e = g * 16 + e16
                for j in range(_D // 16):
                    sl = pl.ds(j * 16, 16)
                    rows[b][e, sl] = rows[b][e, sl] * s
            return carry2

        lax.fori_loop(0, _CHUNK // 16, scale, 0, unroll=True)

    sem_c = (sem_c0, sem_c1)
    wd_v = (wd0, wd1)

    def commit_issue(b):
        pltpu.async_copy(rows[b], msg_acc.at[s_v[b]], sem_c[b], add=True)
        if with_degree:
            for j in range(_CHUNK // 16):
                sl = pl.ds(j * 16, 16)
                wd_v[b][sl] = lax.bitcast_convert_type(
                    m_v[b][pl.ds(2 * _CHUNK + j * 16, 16)], jnp.float32)
            pltpu.async_copy(wd_v[b], deg_acc.at[s_v[b]], sem_c[b], add=True)

    def commit_wait(b):
        pltpu.make_async_copy(rows[b], msg_acc.at[s_v[b]], sem_c[b]).wait()
        if with_degree:
            pltpu.make_async_copy(wd_v[b], deg_acc.at[s_v[b]], sem_c[b]).wait()

    pltpu.sync_copy(meta_hbm.at[sid, 0], m0)
    transform(0)
    gather_issue(0)
    idx_issue(1, 1)

    def pair(cc, carry):
        for b in range(2):
            c = cc * 2 + b
            q = 1 - b
            gather_wait(b)

            @pl.when(c + 1 < _N_CHUNKS)
            def _():
                idx_wait(c + 1, q)

                @pl.when(c >= 1)
                def _():
                    commit_wait(q)

                transform(q)
                gather_issue(q)

            scale_rows(b)
            commit_issue(b)

            @pl.when(c + 2 < _N_CHUNKS)
            def _():
                idx_issue(c + 2, b)
        return carry

    lax.fori_loop(0, _N_CHUNKS // 2, pair, 0)

    commit_wait(0)
    commit_wait(1)
    plsc.subcore_barrier()

    obase = cid * _HALF + rbase
    for k in range(_ROWS_PER_TILE // _CHUNK):
        pltpu.sync_copy(msg_acc.at[pl.ds(rbase + k * _CHUNK, _CHUNK)],
                        msg_out.at[pl.ds(obase + k * _CHUNK, _CHUNK)])
    if with_degree:
        for k in range(_HALF // 128):
            @pl.when(sid == k % _NS)
            def _():
                pltpu.sync_copy(deg_acc.at[pl.ds(k * 128, 128)],
                                deg_out.at[pl.ds(cid * _HALF + k * 128, 128)])


@functools.cache
def _sc_kernels():
    mesh = plsc.VectorSubcoreMesh(core_axis_name="c", subcore_axis_name="s",
                                  num_cores=_NC)

    k2 = functools.partial(
        pl.kernel,
        mesh=mesh,
        out_type=jax.ShapeDtypeStruct((_N_INTER,), jnp.float32),
        scratch_types=[
            pltpu.VMEM((_CHUNK,), jnp.int32),
            pltpu.VMEM((_CHUNK,), jnp.int32),
            pltpu.VMEM((_CHUNK,), jnp.int32),
            pltpu.VMEM((_CHUNK,), jnp.int32),
            pltpu.VMEM((_CHUNK,), jnp.int32),
            pltpu.VMEM((_CHUNK,), jnp.int32),
            pltpu.VMEM((_CHUNK,), jnp.float32),
            pltpu.VMEM((_CHUNK,), jnp.float32),
            pltpu.SemaphoreType.DMA,
            pltpu.SemaphoreType.DMA,
            pltpu.SemaphoreType.DMA,
            pltpu.SemaphoreType.DMA,
        ],
    )(_k2_body)

    def make_k3(with_degree):
        out_type = [
            jax.ShapeDtypeStruct((_N_PAD, _D), jnp.float32),
            jax.ShapeDtypeStruct((_N_PAD,), jnp.float32),
        ]
        return functools.partial(
            pl.kernel,
            mesh=mesh,
            out_type=out_type,
            scratch_types=(
                [pltpu.VMEM((3 * _CHUNK,), jnp.int32) for _ in range(2)]
                + [pltpu.VMEM((_CHUNK,), jnp.int32) for _ in range(4)]
                + [pltpu.VMEM((_CHUNK,), jnp.float32) for _ in range(2)]
                + [
                    pltpu.VMEM((_CHUNK, _D), jnp.float32),
                    pltpu.VMEM((_CHUNK, _D), jnp.float32),
                    pltpu.VMEM_SHARED((_HALF, _D), jnp.float32),
                    pltpu.VMEM_SHARED((_HALF,), jnp.float32),
                    pltpu.SemaphoreType.DMA,
                    pltpu.SemaphoreType.DMA,
                    pltpu.SemaphoreType.DMA,
                    pltpu.SemaphoreType.DMA,
                    pltpu.SemaphoreType.DMA,
                    pltpu.SemaphoreType.DMA,
                ]
            ),
        )(functools.partial(_k3_body, with_degree))

    return k2, make_k3(True), make_k3(False)


def _k4a_body(deg_ref, msg_ref, w_ref, b_ref, cur_ref, dinv_ref):
    deg = deg_ref[0, :] + 1e-08
    dinv = lax.rsqrt(deg)
    dinv = jnp.where(jnp.isinf(dinv), 0.0, dinv)
    m = msg_ref[...] * dinv[:, None]
    cur = lax.dot_general(m, w_ref[...], (((1,), (1,)), ((), ())),
                          preferred_element_type=jnp.float32)
    cur_ref[...] = jnp.maximum(cur + b_ref[...], 0.0)
    dinv_ref[...] = dinv[None, :]


def _k4a_layer1(deg, msg, W, b):
    blk = 1024
    grid = (_N_PAD // blk,)
    return pl.pallas_call(
        _k4a_body,
        grid=grid,
        in_specs=[
            pl.BlockSpec((1, blk), lambda r: (0, r)),
            pl.BlockSpec((blk, _D), lambda r: (r, 0)),
            pl.BlockSpec((_D, _D), lambda r: (0, 0)),
            pl.BlockSpec((1, _D), lambda r: (0, 0)),
        ],
        out_specs=(
            pl.BlockSpec((blk, _D), lambda r: (r, 0)),
            pl.BlockSpec((1, blk), lambda r: (0, r)),
        ),
        out_shape=(
            jax.ShapeDtypeStruct((_N_PAD, _D), jnp.float32),
            jax.ShapeDtypeStruct((1, _N_PAD), jnp.float32),
        ),
    )(deg.reshape(1, _N_PAD), msg, W, b)


def _k4b_body(dinv_ref, msg_ref, w_ref, b_ref, ego_ref, cur1_ref, out_ref):
    m = msg_ref[...] * dinv_ref[0, :][:, None]
    cur2 = lax.dot_general(m, w_ref[...], (((1,), (1,)), ((), ())),
                           preferred_element_type=jnp.float32)
    cur2 = jnp.maximum(cur2 + b_ref[...], 0.0)
    out_ref[...] = (ego_ref[...] + cur1_ref[...] + cur2) * (1.0 / 3.0)


def _k4b_layer2(dinv, msg, W, b, ego_pad, cur1):
    blk = 1024
    grid = (_N_PAD // blk,)
    return pl.pallas_call(
        _k4b_body,
        grid=grid,
        in_specs=[
            pl.BlockSpec((1, blk), lambda r: (0, r)),
            pl.BlockSpec((blk, _D), lambda r: (r, 0)),
            pl.BlockSpec((_D, _D), lambda r: (0, 0)),
            pl.BlockSpec((1, _D), lambda r: (0, 0)),
            pl.BlockSpec((blk, _D), lambda r: (r, 0)),
            pl.BlockSpec((blk, _D), lambda r: (r, 0)),
        ],
        out_specs=pl.BlockSpec((blk, _D), lambda r: (r, 0)),
        out_shape=jax.ShapeDtypeStruct((_N_PAD, _D), jnp.float32),
    )(dinv, msg, W, b, ego_pad, cur1)


def kernel(ego_embeddings, denoise_user_ids, denoise_item_ids, denoise_treatments, alpha, beta, W1, b1, W2, b2):
    uid = denoise_user_ids.astype(jnp.int32)
    iid = denoise_item_ids.astype(jnp.int32)
    ab = jnp.stack([alpha, beta]).reshape(1, 2).astype(jnp.float32)

    k2_gather_s, k3_msg_deg, k3_msg = _sc_kernels()

    s_mat = _k1_sim(ego_embeddings[:_N_USERS], ego_embeddings[_N_USERS:], ab)
    s_edge = k2_gather_s(s_mat.reshape(-1), uid, iid)
    ipw2, loss = _k5_edge_elem(s_edge, denoise_treatments)

    eshape = (_NS, _N_CHUNKS, _CHUNK)
    meta = jnp.concatenate(
        [uid.reshape(eshape), iid.reshape(eshape),
         jax.lax.bitcast_convert_type(ipw2, jnp.int32).reshape(eshape)], axis=2)

    zpad = jnp.zeros((_HALF - _N_USERS, _D), jnp.float32)
    ego_pad = jnp.concatenate(
        [ego_embeddings[:_N_USERS], zpad, ego_embeddings[_N_USERS:], zpad], axis=0)
    msg, deg = k3_msg_deg(ego_pad, meta)
    cur1, dinv = _k4a_layer1(deg, msg, W1, b1.reshape(1, _D))
    msg2, _ = k3_msg(cur1, meta)
    den_pad = _k4b_layer2(dinv, msg2, W2, b2.reshape(1, _D), ego_pad, cur1)
    den = jnp.concatenate(
        [den_pad[:_N_USERS], den_pad[_HALF:_HALF + _N_ITEMS]], axis=0)
    return (den, loss.reshape(()))


# speedup vs baseline: 12.4169x; 1.0764x over previous
import functools

import jax
import jax.numpy as jnp
from jax import lax
from jax.experimental import pallas as pl
from jax.experimental.pallas import tpu as pltpu
from jax.experimental.pallas import tpu_sc as plsc

_N_USERS = 5000
_N_ITEMS = 5000
_N_NODES = _N_USERS + _N_ITEMS
_N_INTER = 320000
_D = 128
_S_COLS = 5120
_N_PAD = 10240

_NC = 2
_NS = 16
_NW = _NC * _NS
_HALF = _N_PAD // 2
_CHUNK = 80
_E_PER_W = _N_INTER // _NW
_K2_CHUNKS = _E_PER_W // _CHUNK
_E_PER_TILE = _N_INTER // _NS
_N_CHUNKS = _E_PER_TILE // _CHUNK
_ROWS_PER_TILE = _HALF // _NS


def _k1_body(u_ref, i_ref, ab_ref, s_ref):
    u = u_ref[...]
    it = i_ref[...]
    u_inv = 1.0 / jnp.maximum(jnp.sqrt(jnp.sum(u * u, axis=1, keepdims=True)), 1e-12)
    i_inv = 1.0 / jnp.maximum(jnp.sqrt(jnp.sum(it * it, axis=1, keepdims=True)), 1e-12)
    un = u * u_inv
    inr = it * i_inv
    s = lax.dot_general(un, inr, (((1,), (1,)), ((), ())),
                        preferred_element_type=jnp.float32)
    s_ref[...] = ab_ref[0, 0] * s + ab_ref[0, 1]


def _k1_sim(u_emb, i_emb, ab):
    blk = 512
    grid = (10, 10)
    return pl.pallas_call(
        _k1_body,
        grid=grid,
        in_specs=[
            pl.BlockSpec((blk, _D), lambda i, j: (i, 0)),
            pl.BlockSpec((blk, _D), lambda i, j: (j, 0)),
            pl.BlockSpec(memory_space=pltpu.SMEM),
        ],
        out_specs=pl.BlockSpec((blk, blk), lambda i, j: (i, j)),
        out_shape=jax.ShapeDtypeStruct((_N_USERS, _S_COLS), jnp.float32),
    )(u_emb, i_emb, ab)


def _k2_body(s_flat, uid_hbm, iid_hbm, s_edge_out,
             u0, u1, i0, i1, f0, f1, sv0, sv1, sem_g0, sem_g1, sem_x0, sem_x1):
    wid = lax.axis_index("c") * _NS + lax.axis_index("s")
    base = wid * _E_PER_W
    u_v = (u0, u1)
    i_v = (i0, i1)
    f_v = (f0, f1)
    s_v = (sv0, sv1)
    sem_g = (sem_g0, sem_g1)
    sem_x = (sem_x0, sem_x1)

    def off(c):
        return base + c * _CHUNK

    def idx_issue(c, b):
        pltpu.async_copy(uid_hbm.at[pl.ds(off(c), _CHUNK)], u_v[b], sem_x[b])
        pltpu.async_copy(iid_hbm.at[pl.ds(off(c), _CHUNK)], i_v[b], sem_x[b])

    def idx_wait(c, b):
        pltpu.make_async_copy(uid_hbm.at[pl.ds(off(c), _CHUNK)], u_v[b], sem_x[b]).wait()
        pltpu.make_async_copy(iid_hbm.at[pl.ds(off(c), _CHUNK)], i_v[b], sem_x[b]).wait()

    def flat(b):
        for j in range(_CHUNK // 16):
            sl = pl.ds(j * 16, 16)
            f_v[b][sl] = u_v[b][sl] * _S_COLS + i_v[b][sl]

    def gather_issue(b):
        pltpu.async_copy(s_flat.at[f_v[b]], s_v[b], sem_g[b])

    def gather_wait(b):
        pltpu.make_async_copy(s_flat.at[f_v[b]], s_v[b], sem_g[b]).wait()

    pltpu.sync_copy(uid_hbm.at[pl.ds(off(0), _CHUNK)], u0)
    pltpu.sync_copy(iid_hbm.at[pl.ds(off(0), _CHUNK)], i0)
    flat(0)
    gather_issue(0)
    idx_issue(1, 1)

    def pair(cc, carry):
        for b in range(2):
            c = cc * 2 + b
            q = 1 - b
            gather_wait(b)

            @pl.when(c + 1 < _K2_CHUNKS)
            def _():
                idx_wait(c + 1, q)
                flat(q)
                gather_issue(q)

            pltpu.sync_copy(s_v[b], s_edge_out.at[pl.ds(off(c), _CHUNK)])

            @pl.when(c + 2 < _K2_CHUNKS)
            def _():
                idx_issue(c + 2, b)
        return carry

    lax.fori_loop(0, _K2_CHUNKS // 2, pair, 0)

    c_last = _K2_CHUNKS - 1
    gather_wait(0)
    pltpu.sync_copy(s_v[0], s_edge_out.at[pl.ds(off(c_last), _CHUNK)])


def _k5_body(s_ref, t_ref, ipw_ref, loss_ref):
    s = s_ref[...]
    t = t_ref[...]
    e = jax.nn.sigmoid(s)
    ll = (t * jnp.log(jnp.clip(e, 1e-12, 1.0))
          + (1.0 - t) * jnp.log(jnp.clip(1.0 - e, 1e-12, 1.0)))
    loss_ref[0, 0] = -jnp.sum(ll) * (1.0 / _N_INTER)
    ipw_ref[...] = t / (e + 1e-08)


def _k5_edge_elem(s_edge, treat):
    shp = (_N_INTER // _D, _D)
    return pl.pallas_call(
        _k5_body,
        out_shape=(
            jax.ShapeDtypeStruct(shp, jnp.float32),
            jax.ShapeDtypeStruct((1, 1), jnp.float32),
        ),
        out_specs=(
            pl.BlockSpec(shp, lambda: (0, 0)),
            pl.BlockSpec(memory_space=pltpu.SMEM),
        ),
    )(s_edge.reshape(shp), treat.reshape(shp))


def _k3_body(with_degree, cur_hbm, meta_hbm, msg_out, deg_out,
             m0, m1, m2, m3, g0, g1, g2, g3, s0, s1, s2, s3,
             wd0, wd1, wd2, wd3, rows0, rows1, rows2, rows3,
             msg_acc, deg_acc,
             sem_r0, sem_r1, sem_r2, sem_r3,
             sem_x0, sem_x1, sem_x2, sem_x3,
             sem_c0, sem_c1, sem_c2, sem_c3):
    cid = lax.axis_index("c")
    sid = lax.axis_index("s")
    rbase = sid * _ROWS_PER_TILE
    m_v = (m0, m1, m2, m3)
    g_v = (g0, g1, g2, g3)
    s_v = (s0, s1, s2, s3)
    rows = (rows0, rows1, rows2, rows3)
    sem_r = (sem_r0, sem_r1, sem_r2, sem_r3)
    sem_x = (sem_x0, sem_x1, sem_x2, sem_x3)

    def zrow(e, carry):
        for j in range(_D // 16):
            rows0[e, pl.ds(j * 16, 16)] = jnp.zeros((16,), jnp.float32)
        return carry

    lax.fori_loop(0, _CHUNK, zrow, 0)
    for k in range(_ROWS_PER_TILE // _CHUNK):
        pltpu.sync_copy(rows0, msg_acc.at[pl.ds(rbase + k * _CHUNK, _CHUNK)])
    for k in range(_HALF // 128):
        @pl.when(sid == k % _NS)
        def _():
            pltpu.sync_copy(rows0.at[0], deg_acc.at[pl.ds(k * 128, 128)])
    plsc.subcore_barrier()

    def idx_issue(c, b):
        pltpu.async_copy(meta_hbm.at[sid, c], m_v[b], sem_x[b])

    def idx_wait(c, b):
        pltpu.make_async_copy(meta_hbm.at[sid, c], m_v[b], sem_x[b]).wait()

    def transform(b):
        @pl.when(cid == 0)
        def _():
            for j in range(_CHUNK // 16):
                sl = pl.ds(j * 16, 16)
                g_v[b][sl] = m_v[b][pl.ds(_CHUNK + j * 16, 16)] + _HALF
                s_v[b][sl] = m_v[b][sl]

        @pl.when(cid == 1)
        def _():
            for j in range(_CHUNK // 16):
                sl = pl.ds(j * 16, 16)
                g_v[b][sl] = m_v[b][sl]
                s_v[b][sl] = m_v[b][pl.ds(_CHUNK + j * 16, 16)]

    def gather_issue(b):
        pltpu.async_copy(cur_hbm.at[g_v[b]], rows[b], sem_r[b])

    def gather_wait(b):
        pltpu.make_async_copy(cur_hbm.at[g_v[b]], rows[b], sem_r[b]).wait()

    def scale_rows(b):
        def scale(g, carry2):
            v = lax.bitcast_convert_type(
                m_v[b][pl.ds(2 * _CHUNK + g * 16, 16)], jnp.float32)
            for e16 in range(16):
                s = v[e16]
                e = g * 16 + e16
                for j in range(_D // 16):
                    sl = pl.ds(j * 16, 16)
                    rows[b][e, sl] = rows[b][e, sl] * s
            return carry2

        lax.fori_loop(0, _CHUNK // 16, scale, 0)

    sem_c = (sem_c0, sem_c1, sem_c2, sem_c3)
    wd_v = (wd0, wd1, wd2, wd3)

    def commit_issue(b):
        pltpu.async_copy(rows[b], msg_acc.at[s_v[b]], sem_c[b], add=True)
        if with_degree:
            for j in range(_CHUNK // 16):
                sl = pl.ds(j * 16, 16)
                wd_v[b][sl] = lax.bitcast_convert_type(
                    m_v[b][pl.ds(2 * _CHUNK + j * 16, 16)], jnp.float32)
            pltpu.async_copy(wd_v[b], deg_acc.at[s_v[b]], sem_c[b], add=True)

    def commit_wait(b):
        pltpu.make_async_copy(rows[b], msg_acc.at[s_v[b]], sem_c[b]).wait()
        if with_degree:
            pltpu.make_async_copy(wd_v[b], deg_acc.at[s_v[b]], sem_c[b]).wait()

    pltpu.sync_copy(meta_hbm.at[sid, 0], m0)
    transform(0)
    gather_issue(0)
    pltpu.sync_copy(meta_hbm.at[sid, 1], m1)
    transform(1)
    gather_issue(1)
    idx_issue(2, 2)

    def quad(cc, carry):
        for k in range(4):
            c = cc * 4 + k
            b = k
            b2 = (k + 2) % 4
            b3 = (k + 3) % 4

            @pl.when(c < _N_CHUNKS)
            def _():
                gather_wait(b)

            @pl.when(c + 2 < _N_CHUNKS)
            def _():
                idx_wait(c + 2, b2)

                @pl.when(c >= 2)
                def _():
                    commit_wait(b2)

                transform(b2)
                gather_issue(b2)

            @pl.when(c < _N_CHUNKS)
            def _():
                scale_rows(b)
                commit_issue(b)

            @pl.when(c + 3 < _N_CHUNKS)
            def _():
                idx_issue(c + 3, b3)
        return carry

    lax.fori_loop(0, (_N_CHUNKS + 3) // 4, quad, 0)

    commit_wait(0)
    commit_wait(1)
    commit_wait(2)
    commit_wait(3)
    plsc.subcore_barrier()

    obase = cid * _HALF + rbase
    for k in range(_ROWS_PER_TILE // _CHUNK):
        pltpu.sync_copy(msg_acc.at[pl.ds(rbase + k * _CHUNK, _CHUNK)],
                        msg_out.at[pl.ds(obase + k * _CHUNK, _CHUNK)])
    if with_degree:
        for k in range(_HALF // 128):
            @pl.when(sid == k % _NS)
            def _():
                pltpu.sync_copy(deg_acc.at[pl.ds(k * 128, 128)],
                                deg_out.at[pl.ds(cid * _HALF + k * 128, 128)])


@functools.cache
def _sc_kernels():
    mesh = plsc.VectorSubcoreMesh(core_axis_name="c", subcore_axis_name="s",
                                  num_cores=_NC)

    k2 = functools.partial(
        pl.kernel,
        mesh=mesh,
        out_type=jax.ShapeDtypeStruct((_N_INTER,), jnp.float32),
        scratch_types=[
            pltpu.VMEM((_CHUNK,), jnp.int32),
            pltpu.VMEM((_CHUNK,), jnp.int32),
            pltpu.VMEM((_CHUNK,), jnp.int32),
            pltpu.VMEM((_CHUNK,), jnp.int32),
            pltpu.VMEM((_CHUNK,), jnp.int32),
            pltpu.VMEM((_CHUNK,), jnp.int32),
            pltpu.VMEM((_CHUNK,), jnp.float32),
            pltpu.VMEM((_CHUNK,), jnp.float32),
            pltpu.SemaphoreType.DMA,
            pltpu.SemaphoreType.DMA,
            pltpu.SemaphoreType.DMA,
            pltpu.SemaphoreType.DMA,
        ],
    )(_k2_body)

    def make_k3(with_degree):
        out_type = [
            jax.ShapeDtypeStruct((_N_PAD, _D), jnp.float32),
            jax.ShapeDtypeStruct((_N_PAD,), jnp.float32),
        ]
        return functools.partial(
            pl.kernel,
            mesh=mesh,
            out_type=out_type,
            scratch_types=(
                [pltpu.VMEM((3 * _CHUNK,), jnp.int32) for _ in range(4)]
                + [pltpu.VMEM((_CHUNK,), jnp.int32) for _ in range(8)]
                + [pltpu.VMEM((_CHUNK,), jnp.float32) for _ in range(4)]
                + [pltpu.VMEM((_CHUNK, _D), jnp.float32) for _ in range(4)]
                + [
                    pltpu.VMEM_SHARED((_HALF, _D), jnp.float32),
                    pltpu.VMEM_SHARED((_HALF,), jnp.float32),
                ]
                + [pltpu.SemaphoreType.DMA for _ in range(12)]
            ),
        )(functools.partial(_k3_body, with_degree))

    return k2, make_k3(True), make_k3(False)


def _k4a_body(deg_ref, msg_ref, w_ref, b_ref, cur_ref, dinv_ref):
    deg = deg_ref[0, :] + 1e-08
    dinv = lax.rsqrt(deg)
    dinv = jnp.where(jnp.isinf(dinv), 0.0, dinv)
    m = msg_ref[...] * dinv[:, None]
    cur = lax.dot_general(m, w_ref[...], (((1,), (1,)), ((), ())),
                          preferred_element_type=jnp.float32)
    cur_ref[...] = jnp.maximum(cur + b_ref[...], 0.0)
    dinv_ref[...] = dinv[None, :]


def _k4a_layer1(deg, msg, W, b):
    blk = 1024
    grid = (_N_PAD // blk,)
    return pl.pallas_call(
        _k4a_body,
        grid=grid,
        in_specs=[
            pl.BlockSpec((1, blk), lambda r: (0, r)),
            pl.BlockSpec((blk, _D), lambda r: (r, 0)),
            pl.BlockSpec((_D, _D), lambda r: (0, 0)),
            pl.BlockSpec((1, _D), lambda r: (0, 0)),
        ],
        out_specs=(
            pl.BlockSpec((blk, _D), lambda r: (r, 0)),
            pl.BlockSpec((1, blk), lambda r: (0, r)),
        ),
        out_shape=(
            jax.ShapeDtypeStruct((_N_PAD, _D), jnp.float32),
            jax.ShapeDtypeStruct((1, _N_PAD), jnp.float32),
        ),
    )(deg.reshape(1, _N_PAD), msg, W, b)


def _k4b_body(dinv_ref, msg_ref, w_ref, b_ref, ego_ref, cur1_ref, out_ref):
    m = msg_ref[...] * dinv_ref[0, :][:, None]
    cur2 = lax.dot_general(m, w_ref[...], (((1,), (1,)), ((), ())),
                           preferred_element_type=jnp.float32)
    cur2 = jnp.maximum(cur2 + b_ref[...], 0.0)
    out_ref[...] = (ego_ref[...] + cur1_ref[...] + cur2) * (1.0 / 3.0)


def _k4b_layer2(dinv, msg, W, b, ego_pad, cur1):
    blk = 1024
    grid = (_N_PAD // blk,)
    return pl.pallas_call(
        _k4b_body,
        grid=grid,
        in_specs=[
            pl.BlockSpec((1, blk), lambda r: (0, r)),
            pl.BlockSpec((blk, _D), lambda r: (r, 0)),
            pl.BlockSpec((_D, _D), lambda r: (0, 0)),
            pl.BlockSpec((1, _D), lambda r: (0, 0)),
            pl.BlockSpec((blk, _D), lambda r: (r, 0)),
            pl.BlockSpec((blk, _D), lambda r: (r, 0)),
        ],
        out_specs=pl.BlockSpec((blk, _D), lambda r: (r, 0)),
        out_shape=jax.ShapeDtypeStruct((_N_PAD, _D), jnp.float32),
    )(dinv, msg, W, b, ego_pad, cur1)


def kernel(ego_embeddings, denoise_user_ids, denoise_item_ids, denoise_treatments, alpha, beta, W1, b1, W2, b2):
    uid = denoise_user_ids.astype(jnp.int32)
    iid = denoise_item_ids.astype(jnp.int32)
    ab = jnp.stack([alpha, beta]).reshape(1, 2).astype(jnp.float32)

    k2_gather_s, k3_msg_deg, k3_msg = _sc_kernels()

    s_mat = _k1_sim(ego_embeddings[:_N_USERS], ego_embeddings[_N_USERS:], ab)
    s_edge = k2_gather_s(s_mat.reshape(-1), uid, iid)
    ipw2, loss = _k5_edge_elem(s_edge, denoise_treatments)

    eshape = (_NS, _N_CHUNKS, _CHUNK)
    meta = jnp.concatenate(
        [uid.reshape(eshape), iid.reshape(eshape),
         jax.lax.bitcast_convert_type(ipw2, jnp.int32).reshape(eshape)], axis=2)

    zpad = jnp.zeros((_HALF - _N_USERS, _D), jnp.float32)
    ego_pad = jnp.concatenate(
        [ego_embeddings[:_N_USERS], zpad, ego_embeddings[_N_USERS:], zpad], axis=0)
    msg, deg = k3_msg_deg(ego_pad, meta)
    cur1, dinv = _k4a_layer1(deg, msg, W1, b1.reshape(1, _D))
    msg2, _ = k3_msg(cur1, meta)
    den_pad = _k4b_layer2(dinv, msg2, W2, b2.reshape(1, _D), ego_pad, cur1)
    den = jnp.concatenate(
        [den_pad[:_N_USERS], den_pad[_HALF:_HALF + _N_ITEMS]], axis=0)
    return (den, loss.reshape(()))
